# Initial kernel scaffold; baseline (speedup 1.0000x reference)
import jax
import jax.numpy as jnp
from jax.experimental import pallas as pl


def kernel(x, edge_index, W1, b1, W2, b2, Wd, perm):
    raise NotImplementedError("write your pallas kernel here")


# speedup vs baseline: 76.2131x; 76.2131x over previous
import functools

import jax
import jax.numpy as jnp
from jax import lax
from jax.experimental import pallas as pl
from jax.experimental.pallas import tpu as pltpu
from jax.experimental.pallas import tpu_sc as plsc

N = 10000
E = 320000
D = 128
NC = 2
NS = 16
L = 16
NPAD = 10240
SLICE = NPAD // NS
NV = SLICE // L
EPC = E // NS
EIT = EPC // L

_f32 = jnp.float32
_i32 = jnp.int32


def _zero_vec(ref, nvregs):
    zero16 = jnp.zeros((L,), _f32)

    def zb(i, carry):
        ref[pl.ds(i * L, L)] = zero16
        return carry

    lax.fori_loop(0, nvregs, zb, 0, unroll=4)


def _reduce_partials(acc, S, red, stage, sid, sbase):
    pltpu.sync_copy(acc, S.at[sid])
    plsc.subcore_barrier()
    _zero_vec(red, NV)

    def rb(k, carry):
        pltpu.sync_copy(S.at[k, pl.ds(sbase, SLICE)], stage)

        def ab(j, carry2):
            red[pl.ds(j * L, L)] = red[pl.ds(j * L, L)] + stage[pl.ds(j * L, L)]
            return carry2

        lax.fori_loop(0, NV, ab, 0, unroll=4)
        return carry

    lax.fori_loop(0, NS, rb, 0)
    plsc.subcore_barrier()


def _publish_full(slice_ref, F, vec_a, sbase):
    pltpu.sync_copy(slice_ref, F.at[pl.ds(sbase, SLICE)])
    plsc.subcore_barrier()
    pltpu.sync_copy(F, vec_a)
    plsc.subcore_barrier()


def _rsqrt16(dv):
    magic = jnp.full((L,), 0x5F3759DF, _i32)
    ii = magic - lax.shift_right_logical(plsc.bitcast(dv, _i32), 1)
    yv = plsc.bitcast(ii, _f32)
    yv = yv * (1.5 - 0.5 * dv * yv * yv)
    yv = yv * (1.5 - 0.5 * dv * yv * yv)
    yv = yv * (1.5 - 0.5 * dv * yv * yv)
    return yv


def _sc_a_body(src_hbm, dst_hbm, dinv_hbm, r_hbm, q_hbm, g_hbm,
               src_v, dst_v, vec_a, acc, red, stage, dinv_sl, a2_sl, out_sl,
               S, F):
    c = lax.axis_index("c")
    sid = lax.axis_index("s")
    ebase = sid * EPC
    sbase = sid * SLICE
    pltpu.sync_copy(src_hbm.at[pl.ds(ebase, EPC)], src_v)
    pltpu.sync_copy(dst_hbm.at[pl.ds(ebase, EPC)], dst_v)

    one16 = jnp.ones((L,), _f32)

    _zero_vec(acc, NPAD // L)

    def deg_b(i, carry):
        di = dst_v[pl.ds(i * L, L)]
        plsc.addupdate_scatter(acc, [di], one16)
        return carry

    lax.fori_loop(0, EIT, deg_b, 0)
    _reduce_partials(acc, S, red, stage, sid, sbase)

    def dv_b(j, carry):
        dinv_sl[pl.ds(j * L, L)] = _rsqrt16(red[pl.ds(j * L, L)] + 1.0)
        return carry

    lax.fori_loop(0, NV, dv_b, 0)
    _publish_full(dinv_sl, F, vec_a, sbase)

    def t_pass(i, carry):
        si = src_v[pl.ds(i * L, L)]
        di = dst_v[pl.ds(i * L, L)]
        vals = plsc.load_gather(vec_a, [di])
        plsc.addupdate_scatter(acc, [si], vals)
        return carry

    @pl.when(c == 0)
    def _core0():
        pltpu.sync_copy(dinv_sl, dinv_hbm.at[pl.ds(sbase, SLICE)])
        _zero_vec(acc, NPAD // L)
        lax.fori_loop(0, EIT, t_pass, 0)
        _reduce_partials(acc, S, red, stage, sid, sbase)

        def rf_b(j, carry):
            dsv = dinv_sl[pl.ds(j * L, L)]
            rr = dsv * (red[pl.ds(j * L, L)] + dsv)
            out_sl[pl.ds(j * L, L)] = rr
            a2_sl[pl.ds(j * L, L)] = dsv * rr
            return carry

        lax.fori_loop(0, NV, rf_b, 0)
        pltpu.sync_copy(out_sl, r_hbm.at[pl.ds(sbase, SLICE)])
        _publish_full(a2_sl, F, vec_a, sbase)
        _zero_vec(acc, NPAD // L)
        lax.fori_loop(0, EIT, t_pass, 0)
        _reduce_partials(acc, S, red, stage, sid, sbase)

        def qf_b(j, carry):
            dsv = dinv_sl[pl.ds(j * L, L)]
            out_sl[pl.ds(j * L, L)] = dsv * (red[pl.ds(j * L, L)] + a2_sl[pl.ds(j * L, L)])
            return carry

        lax.fori_loop(0, NV, qf_b, 0)
        pltpu.sync_copy(out_sl, q_hbm.at[pl.ds(sbase, SLICE)])

    @pl.when(c == 1)
    def _core1():
        _zero_vec(acc, NPAD // L)

        def g_b(i, carry):
            si = src_v[pl.ds(i * L, L)]
            di = dst_v[pl.ds(i * L, L)]
            vals = plsc.load_gather(vec_a, [si])
            plsc.addupdate_scatter(acc, [di], vals)
            return carry

        lax.fori_loop(0, EIT, g_b, 0)
        _reduce_partials(acc, S, red, stage, sid, sbase)

        def gf_b(j, carry):
            dsv = dinv_sl[pl.ds(j * L, L)]
            out_sl[pl.ds(j * L, L)] = dsv * (red[pl.ds(j * L, L)] + dsv)
            return carry

        lax.fori_loop(0, NV, gf_b, 0)
        pltpu.sync_copy(out_sl, g_hbm.at[pl.ds(sbase, SLICE)])


def _sc_c_body(src_hbm, dst_hbm, y_hbm, perm_hbm, dinv_hbm, z_hbm,
               src_v, dst_v, vec_a, acc, red, stage, dinv_sl, a_sl, b_sl,
               perm_sl, S, F):
    c = lax.axis_index("c")
    sid = lax.axis_index("s")
    ebase = sid * EPC
    sbase = sid * SLICE
    pltpu.sync_copy(src_hbm.at[pl.ds(ebase, EPC)], src_v)
    pltpu.sync_copy(dst_hbm.at[pl.ds(ebase, EPC)], dst_v)
    pltpu.sync_copy(dinv_hbm.at[pl.ds(sbase, SLICE)], dinv_sl)

    @pl.when(c == 0)
    def _():
        pltpu.sync_copy(y_hbm.at[pl.ds(sbase, SLICE)], stage)

        def ab(j, carry):
            a_sl[pl.ds(j * L, L)] = dinv_sl[pl.ds(j * L, L)] * stage[pl.ds(j * L, L)]
            return carry

        lax.fori_loop(0, NV, ab, 0)

    @pl.when(c == 1)
    def _():
        pltpu.sync_copy(y_hbm, vec_a)
        pltpu.sync_copy(perm_hbm.at[pl.ds(sbase, SLICE)], perm_sl)

        def pb(j, carry):
            pv = perm_sl[pl.ds(j * L, L)]
            yv = plsc.load_gather(vec_a, [pv])
            a_sl[pl.ds(j * L, L)] = dinv_sl[pl.ds(j * L, L)] * yv
            return carry

        lax.fori_loop(0, NV, pb, 0)

    _publish_full(a_sl, F, vec_a, sbase)

    def f_pass(i, carry):
        si = src_v[pl.ds(i * L, L)]
        di = dst_v[pl.ds(i * L, L)]
        vals = plsc.load_gather(vec_a, [si])
        plsc.addupdate_scatter(acc, [di], vals)
        return carry

    _zero_vec(acc, NPAD // L)
    lax.fori_loop(0, EIT, f_pass, 0)
    _reduce_partials(acc, S, red, stage, sid, sbase)

    def bf_b(j, carry):
        dsv = dinv_sl[pl.ds(j * L, L)]
        b_sl[pl.ds(j * L, L)] = dsv * dsv * (red[pl.ds(j * L, L)] + a_sl[pl.ds(j * L, L)])
        return carry

    lax.fori_loop(0, NV, bf_b, 0)
    _publish_full(b_sl, F, vec_a, sbase)
    _zero_vec(acc, NPAD // L)
    lax.fori_loop(0, EIT, f_pass, 0)
    _reduce_partials(acc, S, red, stage, sid, sbase)

    def zf_b(j, carry):
        dsv = dinv_sl[pl.ds(j * L, L)]
        stage[pl.ds(j * L, L)] = dsv * (red[pl.ds(j * L, L)] + b_sl[pl.ds(j * L, L)])
        return carry

    lax.fori_loop(0, NV, zf_b, 0)
    pltpu.sync_copy(stage, z_hbm.at[c, pl.ds(sbase, SLICE)])


def _tc_b_body(x_ref, q_ref, r_ref, w1_ref, w2_ref, wd_ref, b1_ref, b2_ref,
               y_ref, c_ref):
    f32 = jnp.float32
    X = x_ref[...]
    q = q_ref[...]
    sum_r = jnp.sum(r_ref[...])
    qx = lax.dot_general(q, X, (((0,), (0,)), ((), ())),
                         preferred_element_type=f32)
    t1 = lax.dot_general(qx, w1_ref[...], (((1,), (1,)), ((), ())),
                         preferred_element_type=f32)
    m = lax.dot_general(t1 * (1.0 / N) + (sum_r / N) * b1_ref[...],
                        w2_ref[...], (((1,), (1,)), ((), ())),
                        preferred_element_type=f32) + b2_ref[...]
    s = jax.nn.sigmoid(m)
    v = lax.dot_general(s, wd_ref[...], (((1,), (0,)), ((), ())),
                        preferred_element_type=f32)
    u = lax.dot_general(v, w2_ref[...], (((1,), (0,)), ((), ())),
                        preferred_element_type=f32)
    w = lax.dot_general(u, w1_ref[...], (((1,), (0,)), ((), ())),
                        preferred_element_type=f32)
    y_ref[...] = lax.dot_general(X, w, (((1,), (1,)), ((), ())),
                                 preferred_element_type=f32)
    c1 = jnp.sum(b1_ref[...] * u)
    c2 = jnp.sum(b2_ref[...] * v)
    lane = lax.broadcasted_iota(jnp.int32, (1, D), 1)
    c_ref[...] = jnp.where(lane == 0, c1, 0.0) + jnp.where(lane == 1, c2, 0.0)


def _tc_d_body(z1_ref, z2_ref, g_ref, c_ref, o_ref):
    c1 = c_ref[0, 0]
    c2 = c_ref[0, 1]
    g = g_ref[...]
    z1 = z1_ref[...] + c1 * g + c2
    z2 = z2_ref[...] + c1 * g + c2
    p1 = jax.nn.sigmoid(z1)
    p2 = jax.nn.sigmoid(z2)
    lp = jnp.maximum(jnp.log(p1), -100.0)
    l1p = jnp.maximum(jnp.log(1.0 - p2), -100.0)
    o_ref[0, 0] = -0.5 * (jnp.mean(lp) + jnp.mean(l1p))


def kernel(x, edge_index, W1, b1, W2, b2, Wd, perm):
    src = edge_index[0]
    dst = edge_index[1]

    mesh = plsc.VectorSubcoreMesh(core_axis_name="c", subcore_axis_name="s",
                                  num_cores=NC, num_subcores=NS)
    vec_t = jax.ShapeDtypeStruct((NPAD,), _f32)

    sc_a = pl.kernel(
        _sc_a_body,
        out_type=(vec_t, vec_t, vec_t, vec_t),
        mesh=mesh,
        compiler_params=pltpu.CompilerParams(needs_layout_passes=False),
        scratch_types=[
            pltpu.VMEM((EPC,), _i32),
            pltpu.VMEM((EPC,), _i32),
            pltpu.VMEM((NPAD,), _f32),
            pltpu.VMEM((NPAD,), _f32),
            pltpu.VMEM((SLICE,), _f32),
            pltpu.VMEM((SLICE,), _f32),
            pltpu.VMEM((SLICE,), _f32),
            pltpu.VMEM((SLICE,), _f32),
            pltpu.VMEM((SLICE,), _f32),
            pltpu.VMEM_SHARED((NS, NPAD), _f32),
            pltpu.VMEM_SHARED((NPAD,), _f32),
        ],
    )
    dinv, r, q, g = sc_a(src, dst)

    y2, cvec = pl.pallas_call(
        _tc_b_body,
        out_shape=[jax.ShapeDtypeStruct((N, 1), _f32),
                   jax.ShapeDtypeStruct((1, D), _f32)],
    )(x, q[:N].reshape(N, 1), r[:N].reshape(N, 1), W1, W2, Wd,
      b1.reshape(1, D), b2.reshape(1, D))

    y_pad = jnp.concatenate([y2[:, 0], jnp.zeros((NPAD - N,), _f32)])
    perm_pad = jnp.concatenate([perm.astype(_i32),
                                jnp.zeros((NPAD - N,), _i32)])

    sc_c = pl.kernel(
        _sc_c_body,
        out_type=jax.ShapeDtypeStruct((NC, NPAD), _f32),
        mesh=mesh,
        compiler_params=pltpu.CompilerParams(needs_layout_passes=False),
        scratch_types=[
            pltpu.VMEM((EPC,), _i32),
            pltpu.VMEM((EPC,), _i32),
            pltpu.VMEM((NPAD,), _f32),
            pltpu.VMEM((NPAD,), _f32),
            pltpu.VMEM((SLICE,), _f32),
            pltpu.VMEM((SLICE,), _f32),
            pltpu.VMEM((SLICE,), _f32),
            pltpu.VMEM((SLICE,), _f32),
            pltpu.VMEM((SLICE,), _f32),
            pltpu.VMEM((SLICE,), _i32),
            pltpu.VMEM_SHARED((NS, NPAD), _f32),
            pltpu.VMEM_SHARED((NPAD,), _f32),
        ],
    )
    z = sc_c(src, dst, y_pad, perm_pad, dinv)

    out = pl.pallas_call(
        _tc_d_body,
        out_shape=jax.ShapeDtypeStruct((1, 1), _f32),
        in_specs=[
            pl.BlockSpec(memory_space=pltpu.VMEM),
            pl.BlockSpec(memory_space=pltpu.VMEM),
            pl.BlockSpec(memory_space=pltpu.VMEM),
            pl.BlockSpec(memory_space=pltpu.SMEM),
        ],
        out_specs=pl.BlockSpec(memory_space=pltpu.SMEM),
    )(z[0, :N].reshape(N, 1), z[1, :N].reshape(N, 1),
      g[:N].reshape(N, 1), cvec)
    return out[0, 0]


# speedup vs baseline: 77.1402x; 1.0122x over previous
import functools

import jax
import jax.numpy as jnp
from jax import lax
from jax.experimental import pallas as pl
from jax.experimental.pallas import tpu as pltpu
from jax.experimental.pallas import tpu_sc as plsc

N = 10000
E = 320000
D = 128
NC = 2
NS = 16
L = 16
NPAD = 10240
SLICE = NPAD // NS
NV = SLICE // L
EPC = E // NS
EIT = EPC // L

_f32 = jnp.float32
_i32 = jnp.int32


def _zero_vec(ref, nvregs):
    zero16 = jnp.zeros((L,), _f32)

    def zb(i, carry):
        ref[pl.ds(i * L, L)] = zero16
        return carry

    lax.fori_loop(0, nvregs, zb, 0, unroll=4)


def _reduce_partials(acc, S, red, stage, sid, sbase):
    pltpu.sync_copy(acc, S.at[sid])
    plsc.subcore_barrier()
    _zero_vec(red, NV)

    def rb(k, carry):
        pltpu.sync_copy(S.at[k, pl.ds(sbase, SLICE)], stage)

        def ab(j, carry2):
            red[pl.ds(j * L, L)] = red[pl.ds(j * L, L)] + stage[pl.ds(j * L, L)]
            return carry2

        lax.fori_loop(0, NV, ab, 0, unroll=4)
        return carry

    lax.fori_loop(0, NS, rb, 0)
    plsc.subcore_barrier()


def _publish_full(slice_ref, F, vec_a, sbase):
    pltpu.sync_copy(slice_ref, F.at[pl.ds(sbase, SLICE)])
    plsc.subcore_barrier()
    pltpu.sync_copy(F, vec_a)
    plsc.subcore_barrier()


def _rsqrt16(dv):
    magic = jnp.full((L,), 0x5F3759DF, _i32)
    ii = magic - lax.shift_right_logical(plsc.bitcast(dv, _i32), 1)
    yv = plsc.bitcast(ii, _f32)
    yv = yv * (1.5 - 0.5 * dv * yv * yv)
    yv = yv * (1.5 - 0.5 * dv * yv * yv)
    yv = yv * (1.5 - 0.5 * dv * yv * yv)
    return yv


def _sc_a_body(src_hbm, dst_hbm, dinv_hbm, r_hbm, q_hbm, g_hbm,
               src_v, dst_v, vec_a, acc, red, stage, dinv_sl, a2_sl, out_sl,
               S, F):
    c = lax.axis_index("c")
    sid = lax.axis_index("s")
    ebase = sid * EPC
    sbase = sid * SLICE
    pltpu.sync_copy(src_hbm.at[pl.ds(ebase, EPC)], src_v)
    pltpu.sync_copy(dst_hbm.at[pl.ds(ebase, EPC)], dst_v)

    one16 = jnp.ones((L,), _f32)

    _zero_vec(acc, NPAD // L)

    def deg_b(i, carry):
        di = dst_v[pl.ds(i * L, L)]
        plsc.addupdate_scatter(acc, [di], one16)
        return carry

    lax.fori_loop(0, EIT, deg_b, 0, unroll=8)
    _reduce_partials(acc, S, red, stage, sid, sbase)

    def dv_b(j, carry):
        dinv_sl[pl.ds(j * L, L)] = _rsqrt16(red[pl.ds(j * L, L)] + 1.0)
        return carry

    lax.fori_loop(0, NV, dv_b, 0)
    _publish_full(dinv_sl, F, vec_a, sbase)

    def t_pass(i, carry):
        si = src_v[pl.ds(i * L, L)]
        di = dst_v[pl.ds(i * L, L)]
        vals = plsc.load_gather(vec_a, [di])
        plsc.addupdate_scatter(acc, [si], vals)
        return carry

    @pl.when(c == 0)
    def _core0():
        pltpu.sync_copy(dinv_sl, dinv_hbm.at[pl.ds(sbase, SLICE)])
        _zero_vec(acc, NPAD // L)
        lax.fori_loop(0, EIT, t_pass, 0, unroll=8)
        _reduce_partials(acc, S, red, stage, sid, sbase)

        def rf_b(j, carry):
            dsv = dinv_sl[pl.ds(j * L, L)]
            rr = dsv * (red[pl.ds(j * L, L)] + dsv)
            out_sl[pl.ds(j * L, L)] = rr
            a2_sl[pl.ds(j * L, L)] = dsv * rr
            return carry

        lax.fori_loop(0, NV, rf_b, 0)
        pltpu.sync_copy(out_sl, r_hbm.at[pl.ds(sbase, SLICE)])
        _publish_full(a2_sl, F, vec_a, sbase)
        _zero_vec(acc, NPAD // L)
        lax.fori_loop(0, EIT, t_pass, 0, unroll=8)
        _reduce_partials(acc, S, red, stage, sid, sbase)

        def qf_b(j, carry):
            dsv = dinv_sl[pl.ds(j * L, L)]
            out_sl[pl.ds(j * L, L)] = dsv * (red[pl.ds(j * L, L)] + a2_sl[pl.ds(j * L, L)])
            return carry

        lax.fori_loop(0, NV, qf_b, 0)
        pltpu.sync_copy(out_sl, q_hbm.at[pl.ds(sbase, SLICE)])

    @pl.when(c == 1)
    def _core1():
        _zero_vec(acc, NPAD // L)

        def g_b(i, carry):
            si = src_v[pl.ds(i * L, L)]
            di = dst_v[pl.ds(i * L, L)]
            vals = plsc.load_gather(vec_a, [si])
            plsc.addupdate_scatter(acc, [di], vals)
            return carry

        lax.fori_loop(0, EIT, g_b, 0, unroll=8)
        _reduce_partials(acc, S, red, stage, sid, sbase)

        def gf_b(j, carry):
            dsv = dinv_sl[pl.ds(j * L, L)]
            out_sl[pl.ds(j * L, L)] = dsv * (red[pl.ds(j * L, L)] + dsv)
            return carry

        lax.fori_loop(0, NV, gf_b, 0)
        pltpu.sync_copy(out_sl, g_hbm.at[pl.ds(sbase, SLICE)])


def _sc_c_body(src_hbm, dst_hbm, y_hbm, perm_hbm, dinv_hbm, z_hbm,
               src_v, dst_v, vec_a, acc, red, stage, dinv_sl, a_sl, b_sl,
               perm_sl, S, F):
    c = lax.axis_index("c")
    sid = lax.axis_index("s")
    ebase = sid * EPC
    sbase = sid * SLICE
    pltpu.sync_copy(src_hbm.at[pl.ds(ebase, EPC)], src_v)
    pltpu.sync_copy(dst_hbm.at[pl.ds(ebase, EPC)], dst_v)
    pltpu.sync_copy(dinv_hbm.at[pl.ds(sbase, SLICE)], dinv_sl)

    @pl.when(c == 0)
    def _():
        pltpu.sync_copy(y_hbm.at[pl.ds(sbase, SLICE)], stage)

        def ab(j, carry):
            a_sl[pl.ds(j * L, L)] = dinv_sl[pl.ds(j * L, L)] * stage[pl.ds(j * L, L)]
            return carry

        lax.fori_loop(0, NV, ab, 0)

    @pl.when(c == 1)
    def _():
        pltpu.sync_copy(y_hbm, vec_a)
        pltpu.sync_copy(perm_hbm.at[pl.ds(sbase, SLICE)], perm_sl)

        def pb(j, carry):
            pv = perm_sl[pl.ds(j * L, L)]
            yv = plsc.load_gather(vec_a, [pv])
            a_sl[pl.ds(j * L, L)] = dinv_sl[pl.ds(j * L, L)] * yv
            return carry

        lax.fori_loop(0, NV, pb, 0)

    _publish_full(a_sl, F, vec_a, sbase)

    def f_pass(i, carry):
        si = src_v[pl.ds(i * L, L)]
        di = dst_v[pl.ds(i * L, L)]
        vals = plsc.load_gather(vec_a, [si])
        plsc.addupdate_scatter(acc, [di], vals)
        return carry

    _zero_vec(acc, NPAD // L)
    lax.fori_loop(0, EIT, f_pass, 0, unroll=8)
    _reduce_partials(acc, S, red, stage, sid, sbase)

    def bf_b(j, carry):
        dsv = dinv_sl[pl.ds(j * L, L)]
        b_sl[pl.ds(j * L, L)] = dsv * dsv * (red[pl.ds(j * L, L)] + a_sl[pl.ds(j * L, L)])
        return carry

    lax.fori_loop(0, NV, bf_b, 0)
    _publish_full(b_sl, F, vec_a, sbase)
    _zero_vec(acc, NPAD // L)
    lax.fori_loop(0, EIT, f_pass, 0, unroll=8)
    _reduce_partials(acc, S, red, stage, sid, sbase)

    def zf_b(j, carry):
        dsv = dinv_sl[pl.ds(j * L, L)]
        stage[pl.ds(j * L, L)] = dsv * (red[pl.ds(j * L, L)] + b_sl[pl.ds(j * L, L)])
        return carry

    lax.fori_loop(0, NV, zf_b, 0)
    pltpu.sync_copy(stage, z_hbm.at[c, pl.ds(sbase, SLICE)])


def _tc_b_body(x_ref, q_ref, r_ref, w1_ref, w2_ref, wd_ref, b1_ref, b2_ref,
               y_ref, c_ref):
    f32 = jnp.float32
    X = x_ref[...]
    q = q_ref[...]
    sum_r = jnp.sum(r_ref[...])
    qx = lax.dot_general(q, X, (((0,), (0,)), ((), ())),
                         preferred_element_type=f32)
    t1 = lax.dot_general(qx, w1_ref[...], (((1,), (1,)), ((), ())),
                         preferred_element_type=f32)
    m = lax.dot_general(t1 * (1.0 / N) + (sum_r / N) * b1_ref[...],
                        w2_ref[...], (((1,), (1,)), ((), ())),
                        preferred_element_type=f32) + b2_ref[...]
    s = jax.nn.sigmoid(m)
    v = lax.dot_general(s, wd_ref[...], (((1,), (0,)), ((), ())),
                        preferred_element_type=f32)
    u = lax.dot_general(v, w2_ref[...], (((1,), (0,)), ((), ())),
                        preferred_element_type=f32)
    w = lax.dot_general(u, w1_ref[...], (((1,), (0,)), ((), ())),
                        preferred_element_type=f32)
    y_ref[...] = lax.dot_general(X, w, (((1,), (1,)), ((), ())),
                                 preferred_element_type=f32)
    c1 = jnp.sum(b1_ref[...] * u)
    c2 = jnp.sum(b2_ref[...] * v)
    lane = lax.broadcasted_iota(jnp.int32, (1, D), 1)
    c_ref[...] = jnp.where(lane == 0, c1, 0.0) + jnp.where(lane == 1, c2, 0.0)


def _tc_d_body(z1_ref, z2_ref, g_ref, c_ref, o_ref):
    c1 = c_ref[0, 0]
    c2 = c_ref[0, 1]
    g = g_ref[...]
    z1 = z1_ref[...] + c1 * g + c2
    z2 = z2_ref[...] + c1 * g + c2
    p1 = jax.nn.sigmoid(z1)
    p2 = jax.nn.sigmoid(z2)
    lp = jnp.maximum(jnp.log(p1), -100.0)
    l1p = jnp.maximum(jnp.log(1.0 - p2), -100.0)
    o_ref[0, 0] = -0.5 * (jnp.mean(lp) + jnp.mean(l1p))


def kernel(x, edge_index, W1, b1, W2, b2, Wd, perm):
    src = edge_index[0]
    dst = edge_index[1]

    mesh = plsc.VectorSubcoreMesh(core_axis_name="c", subcore_axis_name="s",
                                  num_cores=NC, num_subcores=NS)
    vec_t = jax.ShapeDtypeStruct((NPAD,), _f32)

    sc_a = pl.kernel(
        _sc_a_body,
        out_type=(vec_t, vec_t, vec_t, vec_t),
        mesh=mesh,
        compiler_params=pltpu.CompilerParams(needs_layout_passes=False),
        scratch_types=[
            pltpu.VMEM((EPC,), _i32),
            pltpu.VMEM((EPC,), _i32),
            pltpu.VMEM((NPAD,), _f32),
            pltpu.VMEM((NPAD,), _f32),
            pltpu.VMEM((SLICE,), _f32),
            pltpu.VMEM((SLICE,), _f32),
            pltpu.VMEM((SLICE,), _f32),
            pltpu.VMEM((SLICE,), _f32),
            pltpu.VMEM((SLICE,), _f32),
            pltpu.VMEM_SHARED((NS, NPAD), _f32),
            pltpu.VMEM_SHARED((NPAD,), _f32),
        ],
    )
    dinv, r, q, g = sc_a(src, dst)

    y2, cvec = pl.pallas_call(
        _tc_b_body,
        out_shape=[jax.ShapeDtypeStruct((N, 1), _f32),
                   jax.ShapeDtypeStruct((1, D), _f32)],
    )(x, q[:N].reshape(N, 1), r[:N].reshape(N, 1), W1, W2, Wd,
      b1.reshape(1, D), b2.reshape(1, D))

    y_pad = jnp.concatenate([y2[:, 0], jnp.zeros((NPAD - N,), _f32)])
    perm_pad = jnp.concatenate([perm.astype(_i32),
                                jnp.zeros((NPAD - N,), _i32)])

    sc_c = pl.kernel(
        _sc_c_body,
        out_type=jax.ShapeDtypeStruct((NC, NPAD), _f32),
        mesh=mesh,
        compiler_params=pltpu.CompilerParams(needs_layout_passes=False),
        scratch_types=[
            pltpu.VMEM((EPC,), _i32),
            pltpu.VMEM((EPC,), _i32),
            pltpu.VMEM((NPAD,), _f32),
            pltpu.VMEM((NPAD,), _f32),
            pltpu.VMEM((SLICE,), _f32),
            pltpu.VMEM((SLICE,), _f32),
            pltpu.VMEM((SLICE,), _f32),
            pltpu.VMEM((SLICE,), _f32),
            pltpu.VMEM((SLICE,), _f32),
            pltpu.VMEM((SLICE,), _i32),
            pltpu.VMEM_SHARED((NS, NPAD), _f32),
            pltpu.VMEM_SHARED((NPAD,), _f32),
        ],
    )
    z = sc_c(src, dst, y_pad, perm_pad, dinv)

    out = pl.pallas_call(
        _tc_d_body,
        out_shape=jax.ShapeDtypeStruct((1, 1), _f32),
        in_specs=[
            pl.BlockSpec(memory_space=pltpu.VMEM),
            pl.BlockSpec(memory_space=pltpu.VMEM),
            pl.BlockSpec(memory_space=pltpu.VMEM),
            pl.BlockSpec(memory_space=pltpu.SMEM),
        ],
        out_specs=pl.BlockSpec(memory_space=pltpu.SMEM),
    )(z[0, :N].reshape(N, 1), z[1, :N].reshape(N, 1),
      g[:N].reshape(N, 1), cvec)
    return out[0, 0]


# speedup vs baseline: 98.2971x; 1.2743x over previous
import functools

import jax
import jax.numpy as jnp
from jax import lax
from jax.experimental import pallas as pl
from jax.experimental.pallas import tpu as pltpu
from jax.experimental.pallas import tpu_sc as plsc

N = 10000
E = 320000
D = 128
NC = 2
NS = 16
L = 16
NPAD = 10240
SLICE = NPAD // NS
NV = SLICE // L
EPC = E // NS
EIT = EPC // L

_f32 = jnp.float32
_i32 = jnp.int32


def _zero_vec(ref, nvregs):
    zero16 = jnp.zeros((L,), _f32)

    def zb(i, carry):
        ref[pl.ds(i * L, L)] = zero16
        return carry

    lax.fori_loop(0, nvregs, zb, 0, unroll=4)


def _reduce_partials(acc, S, red, stage, sid, sbase):
    pltpu.sync_copy(acc, S.at[sid])
    plsc.subcore_barrier()
    _zero_vec(red, NV)

    def rb(k, carry):
        pltpu.sync_copy(S.at[k, pl.ds(sbase, SLICE)], stage)

        def ab(j, carry2):
            red[pl.ds(j * L, L)] = red[pl.ds(j * L, L)] + stage[pl.ds(j * L, L)]
            return carry2

        lax.fori_loop(0, NV, ab, 0, unroll=4)
        return carry

    lax.fori_loop(0, NS, rb, 0)
    plsc.subcore_barrier()


def _publish_full(slice_ref, F, vec_a, sbase):
    pltpu.sync_copy(slice_ref, F.at[pl.ds(sbase, SLICE)])
    plsc.subcore_barrier()
    pltpu.sync_copy(F, vec_a)
    plsc.subcore_barrier()


def _rsqrt16(dv):
    magic = jnp.full((L,), 0x5F3759DF, _i32)
    ii = magic - lax.shift_right_logical(plsc.bitcast(dv, _i32), 1)
    yv = plsc.bitcast(ii, _f32)
    yv = yv * (1.5 - 0.5 * dv * yv * yv)
    yv = yv * (1.5 - 0.5 * dv * yv * yv)
    yv = yv * (1.5 - 0.5 * dv * yv * yv)
    return yv


def _sc_a_body(src_hbm, dst_hbm, dinv_hbm, r_hbm, q_hbm, g_hbm,
               src_v, dst_v, vec_a, acc, red, stage, dinv_sl, a2_sl, out_sl,
               S, F):
    c = lax.axis_index("c")
    sid = lax.axis_index("s")
    ebase = sid * EPC
    sbase = sid * SLICE
    pltpu.sync_copy(src_hbm.at[pl.ds(ebase, EPC)], src_v)
    pltpu.sync_copy(dst_hbm.at[pl.ds(ebase, EPC)], dst_v)

    one16 = jnp.ones((L,), _f32)

    _zero_vec(acc, NPAD // L)

    @plsc.parallel_loop(0, EIT, 1, unroll=8)
    def _deg_b(i):
        di = dst_v[pl.ds(i * L, L)]
        plsc.addupdate_scatter(acc, [di], one16)
    _reduce_partials(acc, S, red, stage, sid, sbase)

    def dv_b(j, carry):
        dinv_sl[pl.ds(j * L, L)] = _rsqrt16(red[pl.ds(j * L, L)] + 1.0)
        return carry

    lax.fori_loop(0, NV, dv_b, 0)
    _publish_full(dinv_sl, F, vec_a, sbase)

    def t_pass_loop():
        @plsc.parallel_loop(0, EIT, 1, unroll=8)
        def _t_b(i):
            si = src_v[pl.ds(i * L, L)]
            di = dst_v[pl.ds(i * L, L)]
            vals = plsc.load_gather(vec_a, [di])
            plsc.addupdate_scatter(acc, [si], vals)

    @pl.when(c == 0)
    def _core0():
        pltpu.sync_copy(dinv_sl, dinv_hbm.at[pl.ds(sbase, SLICE)])
        _zero_vec(acc, NPAD // L)
        t_pass_loop()
        _reduce_partials(acc, S, red, stage, sid, sbase)

        def rf_b(j, carry):
            dsv = dinv_sl[pl.ds(j * L, L)]
            rr = dsv * (red[pl.ds(j * L, L)] + dsv)
            out_sl[pl.ds(j * L, L)] = rr
            a2_sl[pl.ds(j * L, L)] = dsv * rr
            return carry

        lax.fori_loop(0, NV, rf_b, 0)
        pltpu.sync_copy(out_sl, r_hbm.at[pl.ds(sbase, SLICE)])
        _publish_full(a2_sl, F, vec_a, sbase)
        _zero_vec(acc, NPAD // L)
        t_pass_loop()
        _reduce_partials(acc, S, red, stage, sid, sbase)

        def qf_b(j, carry):
            dsv = dinv_sl[pl.ds(j * L, L)]
            out_sl[pl.ds(j * L, L)] = dsv * (red[pl.ds(j * L, L)] + a2_sl[pl.ds(j * L, L)])
            return carry

        lax.fori_loop(0, NV, qf_b, 0)
        pltpu.sync_copy(out_sl, q_hbm.at[pl.ds(sbase, SLICE)])

    @pl.when(c == 1)
    def _core1():
        _zero_vec(acc, NPAD // L)

        @plsc.parallel_loop(0, EIT, 1, unroll=8)
        def _g_b(i):
            si = src_v[pl.ds(i * L, L)]
            di = dst_v[pl.ds(i * L, L)]
            vals = plsc.load_gather(vec_a, [si])
            plsc.addupdate_scatter(acc, [di], vals)
        _reduce_partials(acc, S, red, stage, sid, sbase)

        def gf_b(j, carry):
            dsv = dinv_sl[pl.ds(j * L, L)]
            out_sl[pl.ds(j * L, L)] = dsv * (red[pl.ds(j * L, L)] + dsv)
            return carry

        lax.fori_loop(0, NV, gf_b, 0)
        pltpu.sync_copy(out_sl, g_hbm.at[pl.ds(sbase, SLICE)])


def _sc_c_body(src_hbm, dst_hbm, y_hbm, perm_hbm, dinv_hbm, z_hbm,
               src_v, dst_v, vec_a, acc, red, stage, dinv_sl, a_sl, b_sl,
               perm_sl, S, F):
    c = lax.axis_index("c")
    sid = lax.axis_index("s")
    ebase = sid * EPC
    sbase = sid * SLICE
    pltpu.sync_copy(src_hbm.at[pl.ds(ebase, EPC)], src_v)
    pltpu.sync_copy(dst_hbm.at[pl.ds(ebase, EPC)], dst_v)
    pltpu.sync_copy(dinv_hbm.at[pl.ds(sbase, SLICE)], dinv_sl)

    @pl.when(c == 0)
    def _():
        pltpu.sync_copy(y_hbm.at[pl.ds(sbase, SLICE)], stage)

        def ab(j, carry):
            a_sl[pl.ds(j * L, L)] = dinv_sl[pl.ds(j * L, L)] * stage[pl.ds(j * L, L)]
            return carry

        lax.fori_loop(0, NV, ab, 0)

    @pl.when(c == 1)
    def _():
        pltpu.sync_copy(y_hbm, vec_a)
        pltpu.sync_copy(perm_hbm.at[pl.ds(sbase, SLICE)], perm_sl)

        def pb(j, carry):
            pv = perm_sl[pl.ds(j * L, L)]
            yv = plsc.load_gather(vec_a, [pv])
            a_sl[pl.ds(j * L, L)] = dinv_sl[pl.ds(j * L, L)] * yv
            return carry

        lax.fori_loop(0, NV, pb, 0)

    _publish_full(a_sl, F, vec_a, sbase)

    def f_pass_loop():
        @plsc.parallel_loop(0, EIT, 1, unroll=8)
        def _f_b(i):
            si = src_v[pl.ds(i * L, L)]
            di = dst_v[pl.ds(i * L, L)]
            vals = plsc.load_gather(vec_a, [si])
            plsc.addupdate_scatter(acc, [di], vals)

    _zero_vec(acc, NPAD // L)
    f_pass_loop()
    _reduce_partials(acc, S, red, stage, sid, sbase)

    def bf_b(j, carry):
        dsv = dinv_sl[pl.ds(j * L, L)]
        b_sl[pl.ds(j * L, L)] = dsv * dsv * (red[pl.ds(j * L, L)] + a_sl[pl.ds(j * L, L)])
        return carry

    lax.fori_loop(0, NV, bf_b, 0)
    _publish_full(b_sl, F, vec_a, sbase)
    _zero_vec(acc, NPAD // L)
    f_pass_loop()
    _reduce_partials(acc, S, red, stage, sid, sbase)

    def zf_b(j, carry):
        dsv = dinv_sl[pl.ds(j * L, L)]
        stage[pl.ds(j * L, L)] = dsv * (red[pl.ds(j * L, L)] + b_sl[pl.ds(j * L, L)])
        return carry

    lax.fori_loop(0, NV, zf_b, 0)
    pltpu.sync_copy(stage, z_hbm.at[c, pl.ds(sbase, SLICE)])


def _tc_b_body(x_ref, q_ref, r_ref, w1_ref, w2_ref, wd_ref, b1_ref, b2_ref,
               y_ref, c_ref):
    f32 = jnp.float32
    X = x_ref[...]
    q = q_ref[...]
    sum_r = jnp.sum(r_ref[...])
    qx = lax.dot_general(q, X, (((0,), (0,)), ((), ())),
                         preferred_element_type=f32)
    t1 = lax.dot_general(qx, w1_ref[...], (((1,), (1,)), ((), ())),
                         preferred_element_type=f32)
    m = lax.dot_general(t1 * (1.0 / N) + (sum_r / N) * b1_ref[...],
                        w2_ref[...], (((1,), (1,)), ((), ())),
                        preferred_element_type=f32) + b2_ref[...]
    s = jax.nn.sigmoid(m)
    v = lax.dot_general(s, wd_ref[...], (((1,), (0,)), ((), ())),
                        preferred_element_type=f32)
    u = lax.dot_general(v, w2_ref[...], (((1,), (0,)), ((), ())),
                        preferred_element_type=f32)
    w = lax.dot_general(u, w1_ref[...], (((1,), (0,)), ((), ())),
                        preferred_element_type=f32)
    y_ref[...] = lax.dot_general(X, w, (((1,), (1,)), ((), ())),
                                 preferred_element_type=f32)
    c1 = jnp.sum(b1_ref[...] * u)
    c2 = jnp.sum(b2_ref[...] * v)
    lane = lax.broadcasted_iota(jnp.int32, (1, D), 1)
    c_ref[...] = jnp.where(lane == 0, c1, 0.0) + jnp.where(lane == 1, c2, 0.0)


def _tc_d_body(z1_ref, z2_ref, g_ref, c_ref, o_ref):
    c1 = c_ref[0, 0]
    c2 = c_ref[0, 1]
    g = g_ref[...]
    z1 = z1_ref[...] + c1 * g + c2
    z2 = z2_ref[...] + c1 * g + c2
    p1 = jax.nn.sigmoid(z1)
    p2 = jax.nn.sigmoid(z2)
    lp = jnp.maximum(jnp.log(p1), -100.0)
    l1p = jnp.maximum(jnp.log(1.0 - p2), -100.0)
    o_ref[0, 0] = -0.5 * (jnp.mean(lp) + jnp.mean(l1p))


def kernel(x, edge_index, W1, b1, W2, b2, Wd, perm):
    src = edge_index[0]
    dst = edge_index[1]

    mesh = plsc.VectorSubcoreMesh(core_axis_name="c", subcore_axis_name="s",
                                  num_cores=NC, num_subcores=NS)
    vec_t = jax.ShapeDtypeStruct((NPAD,), _f32)

    sc_a = pl.kernel(
        _sc_a_body,
        out_type=(vec_t, vec_t, vec_t, vec_t),
        mesh=mesh,
        compiler_params=pltpu.CompilerParams(needs_layout_passes=False),
        scratch_types=[
            pltpu.VMEM((EPC,), _i32),
            pltpu.VMEM((EPC,), _i32),
            pltpu.VMEM((NPAD,), _f32),
            pltpu.VMEM((NPAD,), _f32),
            pltpu.VMEM((SLICE,), _f32),
            pltpu.VMEM((SLICE,), _f32),
            pltpu.VMEM((SLICE,), _f32),
            pltpu.VMEM((SLICE,), _f32),
            pltpu.VMEM((SLICE,), _f32),
            pltpu.VMEM_SHARED((NS, NPAD), _f32),
            pltpu.VMEM_SHARED((NPAD,), _f32),
        ],
    )
    dinv, r, q, g = sc_a(src, dst)

    y2, cvec = pl.pallas_call(
        _tc_b_body,
        out_shape=[jax.ShapeDtypeStruct((N, 1), _f32),
                   jax.ShapeDtypeStruct((1, D), _f32)],
    )(x, q[:N].reshape(N, 1), r[:N].reshape(N, 1), W1, W2, Wd,
      b1.reshape(1, D), b2.reshape(1, D))

    y_pad = jnp.concatenate([y2[:, 0], jnp.zeros((NPAD - N,), _f32)])
    perm_pad = jnp.concatenate([perm.astype(_i32),
                                jnp.zeros((NPAD - N,), _i32)])

    sc_c = pl.kernel(
        _sc_c_body,
        out_type=jax.ShapeDtypeStruct((NC, NPAD), _f32),
        mesh=mesh,
        compiler_params=pltpu.CompilerParams(needs_layout_passes=False),
        scratch_types=[
            pltpu.VMEM((EPC,), _i32),
            pltpu.VMEM((EPC,), _i32),
            pltpu.VMEM((NPAD,), _f32),
            pltpu.VMEM((NPAD,), _f32),
            pltpu.VMEM((SLICE,), _f32),
            pltpu.VMEM((SLICE,), _f32),
            pltpu.VMEM((SLICE,), _f32),
            pltpu.VMEM((SLICE,), _f32),
            pltpu.VMEM((SLICE,), _f32),
            pltpu.VMEM((SLICE,), _i32),
            pltpu.VMEM_SHARED((NS, NPAD), _f32),
            pltpu.VMEM_SHARED((NPAD,), _f32),
        ],
    )
    z = sc_c(src, dst, y_pad, perm_pad, dinv)

    out = pl.pallas_call(
        _tc_d_body,
        out_shape=jax.ShapeDtypeStruct((1, 1), _f32),
        in_specs=[
            pl.BlockSpec(memory_space=pltpu.VMEM),
            pl.BlockSpec(memory_space=pltpu.VMEM),
            pl.BlockSpec(memory_space=pltpu.VMEM),
            pl.BlockSpec(memory_space=pltpu.SMEM),
        ],
        out_specs=pl.BlockSpec(memory_space=pltpu.SMEM),
    )(z[0, :N].reshape(N, 1), z[1, :N].reshape(N, 1),
      g[:N].reshape(N, 1), cvec)
    return out[0, 0]


# speedup vs baseline: 112.4817x; 1.1443x over previous
import jax
import jax.numpy as jnp
from jax import lax
from jax.experimental import pallas as pl
from jax.experimental.pallas import tpu as pltpu
from jax.experimental.pallas import tpu_sc as plsc

N = 10000
E = 320000
D = 128
NS = 16
L = 16
NPAD = 10240
SLICE = NPAD // NS
NV = SLICE // L
EPC = E // NS
EIT = EPC // L

_f32 = jnp.float32
_i32 = jnp.int32


def _zero_vec(ref, base, nvregs):
    zero16 = jnp.zeros((L,), _f32)

    @plsc.parallel_loop(0, nvregs, 1, unroll=8)
    def _zb(i):
        ref[pl.ds(base + i * L, L)] = zero16


def _reduce_partials(acc, nwords, S, red, stage16, col, sid, sbase):
    pltpu.sync_copy(S.at[:, pl.ds(col * NPAD + sbase, SLICE)], stage16)

    @plsc.parallel_loop(0, NV, 1, unroll=2)
    def _ab(j):
        t = stage16[0, pl.ds(j * L, L)]
        for k in range(1, NS):
            t = t + stage16[k, pl.ds(j * L, L)]
        red[pl.ds(j * L, L)] = t


def _rsqrt16(dv):
    magic = jnp.full((L,), 0x5F3759DF, _i32)
    ii = magic - lax.shift_right_logical(plsc.bitcast(dv, _i32), 1)
    yv = plsc.bitcast(ii, _f32)
    yv = yv * (1.5 - 0.5 * dv * yv * yv)
    yv = yv * (1.5 - 0.5 * dv * yv * yv)
    yv = yv * (1.5 - 0.5 * dv * yv * yv)
    return yv


def _sc_a_body(src_hbm, dst_hbm, dinv_hbm, r_hbm, q_hbm,
               src_v, dst_v, vec_a, acc, red, stage16, dinv_sl, a2_sl, out_sl,
               S, F):
    sid = lax.axis_index("s")
    ebase = sid * EPC
    sbase = sid * SLICE
    pltpu.sync_copy(src_hbm.at[pl.ds(ebase, EPC)], src_v)
    pltpu.sync_copy(dst_hbm.at[pl.ds(ebase, EPC)], dst_v)

    one16 = jnp.ones((L,), _f32)

    _zero_vec(acc, 0, NPAD // L)

    @plsc.parallel_loop(0, EIT, 1, unroll=8)
    def _deg_b(i):
        di = dst_v[pl.ds(i * L, L)]
        plsc.addupdate_scatter(acc, [di], one16)

    pltpu.sync_copy(acc, S.at[sid])
    plsc.subcore_barrier()
    _reduce_partials(acc, NPAD, S, red, stage16, 0, sid, sbase)

    @plsc.parallel_loop(0, NV, 1)
    def _dv_b(j):
        dinv_sl[pl.ds(j * L, L)] = _rsqrt16(red[pl.ds(j * L, L)] + 1.0)

    plsc.subcore_barrier()
    pltpu.sync_copy(dinv_sl, F.at[pl.ds(sbase, SLICE)])
    pltpu.sync_copy(dinv_sl, dinv_hbm.at[pl.ds(sbase, SLICE)])
    plsc.subcore_barrier()
    pltpu.sync_copy(F, vec_a)

    def t_pass_loop():
        @plsc.parallel_loop(0, EIT, 1, unroll=8)
        def _t_b(i):
            si = src_v[pl.ds(i * L, L)]
            di = dst_v[pl.ds(i * L, L)]
            vals = plsc.load_gather(vec_a, [di])
            plsc.addupdate_scatter(acc, [si], vals)

    _zero_vec(acc, 0, NPAD // L)
    t_pass_loop()
    pltpu.sync_copy(acc, S.at[sid])
    plsc.subcore_barrier()
    _reduce_partials(acc, NPAD, S, red, stage16, 0, sid, sbase)

    @plsc.parallel_loop(0, NV, 1)
    def _rf_b(j):
        dsv = dinv_sl[pl.ds(j * L, L)]
        rr = dsv * (red[pl.ds(j * L, L)] + dsv)
        out_sl[pl.ds(j * L, L)] = rr
        a2_sl[pl.ds(j * L, L)] = dsv * rr

    pltpu.sync_copy(out_sl, r_hbm.at[pl.ds(sbase, SLICE)])
    plsc.subcore_barrier()
    pltpu.sync_copy(a2_sl, F.at[pl.ds(sbase, SLICE)])
    plsc.subcore_barrier()
    pltpu.sync_copy(F, vec_a)

    _zero_vec(acc, 0, NPAD // L)
    t_pass_loop()
    pltpu.sync_copy(acc, S.at[sid])
    plsc.subcore_barrier()
    _reduce_partials(acc, NPAD, S, red, stage16, 0, sid, sbase)

    @plsc.parallel_loop(0, NV, 1)
    def _qf_b(j):
        dsv = dinv_sl[pl.ds(j * L, L)]
        out_sl[pl.ds(j * L, L)] = dsv * (red[pl.ds(j * L, L)] + a2_sl[pl.ds(j * L, L)])

    pltpu.sync_copy(out_sl, q_hbm.at[pl.ds(sbase, SLICE)])


def _sc_c_body(src_hbm, dst_hbm, y_hbm, perm_hbm, dinv_hbm, z_hbm, g_hbm,
               src_v, dst_v, vecs, acc, red, stage16, dinv_sl, perm_sl,
               b1_sl, b2_sl, out_sl, S, F2):
    sid = lax.axis_index("s")
    ebase = sid * EPC
    sbase = sid * SLICE
    pltpu.sync_copy(src_hbm.at[pl.ds(ebase, EPC)], src_v)
    pltpu.sync_copy(dst_hbm.at[pl.ds(ebase, EPC)], dst_v)
    pltpu.sync_copy(dinv_hbm.at[pl.ds(sbase, SLICE)], dinv_sl)
    pltpu.sync_copy(perm_hbm.at[pl.ds(sbase, SLICE)], perm_sl)
    pltpu.sync_copy(y_hbm, vecs.at[pl.ds(0, NPAD)])
    pltpu.sync_copy(dinv_hbm, vecs.at[pl.ds(2 * NPAD, NPAD)])

    @plsc.parallel_loop(0, NV, 1, unroll=4)
    def _ac_b(j):
        pv = perm_sl[pl.ds(j * L, L)]
        yv = plsc.load_gather(vecs, [pv])
        out_sl[pl.ds(j * L, L)] = dinv_sl[pl.ds(j * L, L)] * yv

    pltpu.sync_copy(out_sl, F2.at[pl.ds(sbase, SLICE)])

    @plsc.parallel_loop(0, NPAD // L, 1, unroll=4)
    def _ar_b(j):
        vecs[pl.ds(j * L, L)] = (vecs[pl.ds(j * L, L)]
                                 * vecs[pl.ds(2 * NPAD + j * L, L)])

    plsc.subcore_barrier()
    pltpu.sync_copy(F2.at[pl.ds(0, NPAD)], vecs.at[pl.ds(NPAD, NPAD)])

    _zero_vec(acc, 0, 3 * NPAD // L)
    off1 = jnp.full((L,), NPAD, _i32)
    off2 = jnp.full((L,), 2 * NPAD, _i32)

    @plsc.parallel_loop(0, EIT, 1, unroll=4)
    def _h1_b(i):
        si = src_v[pl.ds(i * L, L)]
        di = dst_v[pl.ds(i * L, L)]
        v0 = plsc.load_gather(vecs, [si])
        v1 = plsc.load_gather(vecs, [si + off1])
        v2 = plsc.load_gather(vecs, [si + off2])
        plsc.addupdate_scatter(acc, [di], v0)
        plsc.addupdate_scatter(acc, [di + off1], v1)
        plsc.addupdate_scatter(acc, [di + off2], v2)

    def col_reduce(col):
        pltpu.sync_copy(acc.at[pl.ds(col * NPAD, NPAD)], S.at[sid])
        plsc.subcore_barrier()
        _reduce_partials(acc, NPAD, S, red, stage16, 0, sid, sbase)
        plsc.subcore_barrier()

    col_reduce(0)

    @plsc.parallel_loop(0, NV, 1)
    def _b1_b(j):
        dsv = dinv_sl[pl.ds(j * L, L)]
        av = vecs[pl.ds(sbase + j * L, L)]
        b1_sl[pl.ds(j * L, L)] = dsv * dsv * (red[pl.ds(j * L, L)] + av)

    col_reduce(1)

    @plsc.parallel_loop(0, NV, 1)
    def _b2_b(j):
        dsv = dinv_sl[pl.ds(j * L, L)]
        av = vecs[pl.ds(NPAD + sbase + j * L, L)]
        b2_sl[pl.ds(j * L, L)] = dsv * dsv * (red[pl.ds(j * L, L)] + av)

    col_reduce(2)

    @plsc.parallel_loop(0, NV, 1)
    def _g_b(j):
        dsv = dinv_sl[pl.ds(j * L, L)]
        out_sl[pl.ds(j * L, L)] = dsv * (red[pl.ds(j * L, L)] + dsv)

    pltpu.sync_copy(out_sl, g_hbm.at[pl.ds(sbase, SLICE)])
    pltpu.sync_copy(b1_sl, F2.at[pl.ds(sbase, SLICE)])
    pltpu.sync_copy(b2_sl, F2.at[pl.ds(NPAD + sbase, SLICE)])
    plsc.subcore_barrier()
    pltpu.sync_copy(F2, vecs.at[pl.ds(0, 2 * NPAD)])

    _zero_vec(acc, 0, 2 * NPAD // L)

    @plsc.parallel_loop(0, EIT, 1, unroll=4)
    def _h2_b(i):
        si = src_v[pl.ds(i * L, L)]
        di = dst_v[pl.ds(i * L, L)]
        v0 = plsc.load_gather(vecs, [si])
        v1 = plsc.load_gather(vecs, [si + off1])
        plsc.addupdate_scatter(acc, [di], v0)
        plsc.addupdate_scatter(acc, [di + off1], v1)

    col_reduce(0)

    @plsc.parallel_loop(0, NV, 1)
    def _z1_b(j):
        dsv = dinv_sl[pl.ds(j * L, L)]
        bv = b1_sl[pl.ds(j * L, L)]
        out_sl[pl.ds(j * L, L)] = dsv * (red[pl.ds(j * L, L)] + bv)

    pltpu.sync_copy(out_sl, z_hbm.at[0, pl.ds(sbase, SLICE)])

    col_reduce(1)

    @plsc.parallel_loop(0, NV, 1)
    def _z2_b(j):
        dsv = dinv_sl[pl.ds(j * L, L)]
        bv = b2_sl[pl.ds(j * L, L)]
        out_sl[pl.ds(j * L, L)] = dsv * (red[pl.ds(j * L, L)] + bv)

    pltpu.sync_copy(out_sl, z_hbm.at[1, pl.ds(sbase, SLICE)])


def _tc_b_body(x_ref, q_ref, r_ref, w1_ref, w2_ref, wd_ref, b1_ref, b2_ref,
               y_ref, c_ref):
    f32 = jnp.float32
    X = x_ref[...]
    q = q_ref[...]
    sum_r = jnp.sum(r_ref[...])
    qx = lax.dot_general(q, X, (((0,), (0,)), ((), ())),
                         preferred_element_type=f32)
    t1 = lax.dot_general(qx, w1_ref[...], (((1,), (1,)), ((), ())),
                         preferred_element_type=f32)
    m = lax.dot_general(t1 * (1.0 / N) + (sum_r / N) * b1_ref[...],
                        w2_ref[...], (((1,), (1,)), ((), ())),
                        preferred_element_type=f32) + b2_ref[...]
    s = jax.nn.sigmoid(m)
    v = lax.dot_general(s, wd_ref[...], (((1,), (0,)), ((), ())),
                        preferred_element_type=f32)
    u = lax.dot_general(v, w2_ref[...], (((1,), (0,)), ((), ())),
                        preferred_element_type=f32)
    w = lax.dot_general(u, w1_ref[...], (((1,), (0,)), ((), ())),
                        preferred_element_type=f32)
    y_ref[0:N, :] = lax.dot_general(X, w, (((1,), (1,)), ((), ())),
                                    preferred_element_type=f32)
    y_ref[N:NPAD, :] = jnp.zeros((NPAD - N, 1), f32)
    c1 = jnp.sum(b1_ref[...] * u)
    c2 = jnp.sum(b2_ref[...] * v)
    lane = lax.broadcasted_iota(jnp.int32, (1, D), 1)
    c_ref[...] = jnp.where(lane == 0, c1, 0.0) + jnp.where(lane == 1, c2, 0.0)


def _tc_d_body(z_ref, g_ref, c_ref, o_ref):
    c1 = c_ref[0, 0]
    c2 = c_ref[0, 1]
    g = g_ref[...]
    z1 = z_ref[0:1, :] + c1 * g + c2
    z2 = z_ref[1:2, :] + c1 * g + c2
    p1 = jax.nn.sigmoid(z1)
    p2 = jax.nn.sigmoid(z2)
    lp = jnp.maximum(jnp.log(p1), -100.0)
    l1p = jnp.maximum(jnp.log(1.0 - p2), -100.0)
    lane = lax.broadcasted_iota(jnp.int32, (1, NPAD), 1)
    valid = lane < N
    s1 = jnp.sum(jnp.where(valid, lp, 0.0))
    s2 = jnp.sum(jnp.where(valid, l1p, 0.0))
    o_ref[0, 0] = -0.5 * (s1 + s2) / N


def kernel(x, edge_index, W1, b1, W2, b2, Wd, perm):
    mesh = plsc.VectorSubcoreMesh(core_axis_name="c", subcore_axis_name="s",
                                  num_cores=1, num_subcores=NS)
    vec_t = jax.ShapeDtypeStruct((NPAD,), _f32)

    sc_a = pl.kernel(
        _sc_a_body,
        out_type=(vec_t, vec_t, vec_t),
        mesh=mesh,
        compiler_params=pltpu.CompilerParams(needs_layout_passes=False),
        scratch_types=[
            pltpu.VMEM((EPC,), _i32),
            pltpu.VMEM((EPC,), _i32),
            pltpu.VMEM((NPAD,), _f32),
            pltpu.VMEM((NPAD,), _f32),
            pltpu.VMEM((SLICE,), _f32),
            pltpu.VMEM((NS, SLICE), _f32),
            pltpu.VMEM((SLICE,), _f32),
            pltpu.VMEM((SLICE,), _f32),
            pltpu.VMEM((SLICE,), _f32),
            pltpu.VMEM_SHARED((NS, NPAD), _f32),
            pltpu.VMEM_SHARED((NPAD,), _f32),
        ],
    )
    dinv, r, q = sc_a(edge_index[0], edge_index[1])

    y2, cvec = pl.pallas_call(
        _tc_b_body,
        out_shape=[jax.ShapeDtypeStruct((NPAD, 1), _f32),
                   jax.ShapeDtypeStruct((1, D), _f32)],
    )(x, q[:N].reshape(N, 1), r[:N].reshape(N, 1), W1, W2, Wd,
      b1.reshape(1, D), b2.reshape(1, D))

    perm_pad = jnp.concatenate([perm.astype(_i32),
                                jnp.zeros((NPAD - N,), _i32)])

    sc_c = pl.kernel(
        _sc_c_body,
        out_type=(jax.ShapeDtypeStruct((2, NPAD), _f32), vec_t),
        mesh=mesh,
        compiler_params=pltpu.CompilerParams(needs_layout_passes=False),
        scratch_types=[
            pltpu.VMEM((EPC,), _i32),
            pltpu.VMEM((EPC,), _i32),
            pltpu.VMEM((3 * NPAD,), _f32),
            pltpu.VMEM((3 * NPAD,), _f32),
            pltpu.VMEM((SLICE,), _f32),
            pltpu.VMEM((NS, SLICE), _f32),
            pltpu.VMEM((SLICE,), _f32),
            pltpu.VMEM((SLICE,), _i32),
            pltpu.VMEM((SLICE,), _f32),
            pltpu.VMEM((SLICE,), _f32),
            pltpu.VMEM((SLICE,), _f32),
            pltpu.VMEM_SHARED((NS, NPAD), _f32),
            pltpu.VMEM_SHARED((2 * NPAD,), _f32),
        ],
    )
    z, g = sc_c(edge_index[0], edge_index[1], y2.reshape(NPAD), perm_pad, dinv)

    out = pl.pallas_call(
        _tc_d_body,
        out_shape=jax.ShapeDtypeStruct((1, 1), _f32),
        in_specs=[
            pl.BlockSpec(memory_space=pltpu.VMEM),
            pl.BlockSpec(memory_space=pltpu.VMEM),
            pl.BlockSpec(memory_space=pltpu.SMEM),
        ],
        out_specs=pl.BlockSpec(memory_space=pltpu.SMEM),
    )(z, g.reshape(1, NPAD), cvec)
    return out[0, 0]


# speedup vs baseline: 126.6659x; 1.1261x over previous
import jax
import jax.numpy as jnp
from jax import lax
from jax.experimental import pallas as pl
from jax.experimental.pallas import tpu as pltpu
from jax.experimental.pallas import tpu_sc as plsc

N = 10000
E = 320000
D = 128
NS = 16
L = 16
NPAD = 10240
SLICE = NPAD // NS
NV = SLICE // L
EPC = E // NS
EIT = EPC // L

_f32 = jnp.float32
_i32 = jnp.int32


def _zero_vec(ref, base, nvregs):
    zero16 = jnp.zeros((L,), _f32)

    @plsc.parallel_loop(0, nvregs, 1, unroll=8)
    def _zb(i):
        ref[pl.ds(base + i * L, L)] = zero16


def _reduce_partials(acc, nwords, S, red, stage16, col, sid, sbase):
    pltpu.sync_copy(S.at[:, pl.ds(col * NPAD + sbase, SLICE)], stage16)

    @plsc.parallel_loop(0, NV, 1, unroll=2)
    def _ab(j):
        t = stage16[0, pl.ds(j * L, L)]
        for k in range(1, NS):
            t = t + stage16[k, pl.ds(j * L, L)]
        red[pl.ds(j * L, L)] = t


def _rsqrt16(dv):
    magic = jnp.full((L,), 0x5F3759DF, _i32)
    ii = magic - lax.shift_right_logical(plsc.bitcast(dv, _i32), 1)
    yv = plsc.bitcast(ii, _f32)
    yv = yv * (1.5 - 0.5 * dv * yv * yv)
    yv = yv * (1.5 - 0.5 * dv * yv * yv)
    yv = yv * (1.5 - 0.5 * dv * yv * yv)
    return yv


def _sc_a_body(ei_hbm, dinv_hbm, r_hbm, q_hbm,
               src_v, dst_v, vec_a, acc, red, stage16, dinv_sl, a2_sl, out_sl,
               S, F, sem1, sem2):
    sid = lax.axis_index("s")
    ebase = sid * EPC
    sbase = sid * SLICE
    cp_s = pltpu.async_copy(ei_hbm.at[pl.ds(ebase, EPC)], src_v, sem1)
    cp_d = pltpu.async_copy(ei_hbm.at[pl.ds(E + ebase, EPC)], dst_v, sem2)

    one16 = jnp.ones((L,), _f32)

    _zero_vec(acc, 0, NPAD // L)
    cp_s.wait()
    cp_d.wait()

    @plsc.parallel_loop(0, EIT, 1, unroll=8)
    def _deg_b(i):
        di = dst_v[pl.ds(i * L, L)]
        plsc.addupdate_scatter(acc, [di], one16)

    pltpu.sync_copy(acc, S.at[sid])
    plsc.subcore_barrier()
    _reduce_partials(acc, NPAD, S, red, stage16, 0, sid, sbase)

    @plsc.parallel_loop(0, NV, 1)
    def _dv_b(j):
        dinv_sl[pl.ds(j * L, L)] = _rsqrt16(red[pl.ds(j * L, L)] + 1.0)

    plsc.subcore_barrier()
    pltpu.sync_copy(dinv_sl, F.at[pl.ds(sbase, SLICE)])
    pltpu.sync_copy(dinv_sl, dinv_hbm.at[pl.ds(sbase, SLICE)])
    plsc.subcore_barrier()
    pltpu.sync_copy(F, vec_a)

    def t_pass_loop():
        @plsc.parallel_loop(0, EIT, 1, unroll=8)
        def _t_b(i):
            si = src_v[pl.ds(i * L, L)]
            di = dst_v[pl.ds(i * L, L)]
            vals = plsc.load_gather(vec_a, [di])
            plsc.addupdate_scatter(acc, [si], vals)

    _zero_vec(acc, 0, NPAD // L)
    t_pass_loop()
    pltpu.sync_copy(acc, S.at[sid])
    plsc.subcore_barrier()
    _reduce_partials(acc, NPAD, S, red, stage16, 0, sid, sbase)

    @plsc.parallel_loop(0, NV, 1)
    def _rf_b(j):
        dsv = dinv_sl[pl.ds(j * L, L)]
        rr = dsv * (red[pl.ds(j * L, L)] + dsv)
        out_sl[pl.ds(j * L, L)] = rr
        a2_sl[pl.ds(j * L, L)] = dsv * rr

    pltpu.sync_copy(out_sl, r_hbm.at[pl.ds(sbase, SLICE)])
    plsc.subcore_barrier()
    pltpu.sync_copy(a2_sl, F.at[pl.ds(sbase, SLICE)])
    plsc.subcore_barrier()
    pltpu.sync_copy(F, vec_a)

    _zero_vec(acc, 0, NPAD // L)
    t_pass_loop()
    pltpu.sync_copy(acc, S.at[sid])
    plsc.subcore_barrier()
    _reduce_partials(acc, NPAD, S, red, stage16, 0, sid, sbase)

    @plsc.parallel_loop(0, NV, 1)
    def _qf_b(j):
        dsv = dinv_sl[pl.ds(j * L, L)]
        out_sl[pl.ds(j * L, L)] = dsv * (red[pl.ds(j * L, L)] + a2_sl[pl.ds(j * L, L)])

    pltpu.sync_copy(out_sl, q_hbm.at[pl.ds(sbase, SLICE)])


def _sc_c_body(ei_hbm, y_hbm, perm_hbm, dinv_hbm, z_hbm, g_hbm,
               src_v, dst_v, vecs, acc, red, stage16, dinv_sl, perm_sl,
               b1_sl, b2_sl, out_sl, S, F2, sem1, sem2):
    sid = lax.axis_index("s")
    ebase = sid * EPC
    sbase = sid * SLICE
    cp_s = pltpu.async_copy(ei_hbm.at[pl.ds(ebase, EPC)], src_v, sem1)
    cp_d = pltpu.async_copy(ei_hbm.at[pl.ds(E + ebase, EPC)], dst_v, sem2)
    pltpu.sync_copy(dinv_hbm.at[pl.ds(sbase, SLICE)], dinv_sl)
    pltpu.sync_copy(perm_hbm.at[pl.ds(sbase, SLICE)], perm_sl)
    pltpu.sync_copy(y_hbm, vecs.at[pl.ds(0, NPAD)])
    pltpu.sync_copy(dinv_hbm, vecs.at[pl.ds(2 * NPAD, NPAD)])

    @plsc.parallel_loop(0, NV, 1, unroll=4)
    def _ac_b(j):
        pv = perm_sl[pl.ds(j * L, L)]
        yv = plsc.load_gather(vecs, [pv])
        out_sl[pl.ds(j * L, L)] = dinv_sl[pl.ds(j * L, L)] * yv

    pltpu.sync_copy(out_sl, F2.at[pl.ds(sbase, SLICE)])

    @plsc.parallel_loop(0, NPAD // L, 1, unroll=4)
    def _ar_b(j):
        vecs[pl.ds(j * L, L)] = (vecs[pl.ds(j * L, L)]
                                 * vecs[pl.ds(2 * NPAD + j * L, L)])

    plsc.subcore_barrier()
    pltpu.sync_copy(F2.at[pl.ds(0, NPAD)], vecs.at[pl.ds(NPAD, NPAD)])

    _zero_vec(acc, 0, 3 * NPAD // L)
    cp_s.wait()
    cp_d.wait()
    off1 = jnp.full((L,), NPAD, _i32)
    off2 = jnp.full((L,), 2 * NPAD, _i32)

    @plsc.parallel_loop(0, EIT, 1, unroll=4)
    def _h1_b(i):
        si = src_v[pl.ds(i * L, L)]
        di = dst_v[pl.ds(i * L, L)]
        v0 = plsc.load_gather(vecs, [si])
        v1 = plsc.load_gather(vecs, [si + off1])
        v2 = plsc.load_gather(vecs, [si + off2])
        plsc.addupdate_scatter(acc, [di], v0)
        plsc.addupdate_scatter(acc, [di + off1], v1)
        plsc.addupdate_scatter(acc, [di + off2], v2)

    def col_reduce(col):
        pltpu.sync_copy(acc.at[pl.ds(col * NPAD, NPAD)], S.at[sid])
        plsc.subcore_barrier()
        _reduce_partials(acc, NPAD, S, red, stage16, 0, sid, sbase)
        plsc.subcore_barrier()

    col_reduce(0)

    @plsc.parallel_loop(0, NV, 1)
    def _b1_b(j):
        dsv = dinv_sl[pl.ds(j * L, L)]
        av = vecs[pl.ds(sbase + j * L, L)]
        b1_sl[pl.ds(j * L, L)] = dsv * dsv * (red[pl.ds(j * L, L)] + av)

    col_reduce(1)

    @plsc.parallel_loop(0, NV, 1)
    def _b2_b(j):
        dsv = dinv_sl[pl.ds(j * L, L)]
        av = vecs[pl.ds(NPAD + sbase + j * L, L)]
        b2_sl[pl.ds(j * L, L)] = dsv * dsv * (red[pl.ds(j * L, L)] + av)

    col_reduce(2)

    @plsc.parallel_loop(0, NV, 1)
    def _g_b(j):
        dsv = dinv_sl[pl.ds(j * L, L)]
        out_sl[pl.ds(j * L, L)] = dsv * (red[pl.ds(j * L, L)] + dsv)

    pltpu.sync_copy(out_sl, g_hbm.at[pl.ds(sbase, SLICE)])
    pltpu.sync_copy(b1_sl, F2.at[pl.ds(sbase, SLICE)])
    pltpu.sync_copy(b2_sl, F2.at[pl.ds(NPAD + sbase, SLICE)])
    plsc.subcore_barrier()
    pltpu.sync_copy(F2, vecs.at[pl.ds(0, 2 * NPAD)])

    _zero_vec(acc, 0, 2 * NPAD // L)

    @plsc.parallel_loop(0, EIT, 1, unroll=4)
    def _h2_b(i):
        si = src_v[pl.ds(i * L, L)]
        di = dst_v[pl.ds(i * L, L)]
        v0 = plsc.load_gather(vecs, [si])
        v1 = plsc.load_gather(vecs, [si + off1])
        plsc.addupdate_scatter(acc, [di], v0)
        plsc.addupdate_scatter(acc, [di + off1], v1)

    col_reduce(0)

    @plsc.parallel_loop(0, NV, 1)
    def _z1_b(j):
        dsv = dinv_sl[pl.ds(j * L, L)]
        bv = b1_sl[pl.ds(j * L, L)]
        out_sl[pl.ds(j * L, L)] = dsv * (red[pl.ds(j * L, L)] + bv)

    pltpu.sync_copy(out_sl, z_hbm.at[0, pl.ds(sbase, SLICE)])

    col_reduce(1)

    @plsc.parallel_loop(0, NV, 1)
    def _z2_b(j):
        dsv = dinv_sl[pl.ds(j * L, L)]
        bv = b2_sl[pl.ds(j * L, L)]
        out_sl[pl.ds(j * L, L)] = dsv * (red[pl.ds(j * L, L)] + bv)

    pltpu.sync_copy(out_sl, z_hbm.at[1, pl.ds(sbase, SLICE)])


def _tc_b_body(x_ref, q_ref, r_ref, w1_ref, w2_ref, wd_ref, b1_ref, b2_ref,
               y_ref, c_ref):
    f32 = jnp.float32
    X = x_ref[...]
    q = q_ref[...]
    sum_r = jnp.sum(r_ref[...])
    qx = lax.dot_general(q, X, (((0,), (0,)), ((), ())),
                         preferred_element_type=f32)
    t1 = lax.dot_general(qx, w1_ref[...], (((1,), (1,)), ((), ())),
                         preferred_element_type=f32)
    m = lax.dot_general(t1 * (1.0 / N) + (sum_r / N) * b1_ref[...],
                        w2_ref[...], (((1,), (1,)), ((), ())),
                        preferred_element_type=f32) + b2_ref[...]
    s = jax.nn.sigmoid(m)
    v = lax.dot_general(s, wd_ref[...], (((1,), (0,)), ((), ())),
                        preferred_element_type=f32)
    u = lax.dot_general(v, w2_ref[...], (((1,), (0,)), ((), ())),
                        preferred_element_type=f32)
    w = lax.dot_general(u, w1_ref[...], (((1,), (0,)), ((), ())),
                        preferred_element_type=f32)
    y_ref[0:N, :] = lax.dot_general(X, w, (((1,), (1,)), ((), ())),
                                    preferred_element_type=f32)
    y_ref[N:NPAD, :] = jnp.zeros((NPAD - N, 1), f32)
    c1 = jnp.sum(b1_ref[...] * u)
    c2 = jnp.sum(b2_ref[...] * v)
    lane = lax.broadcasted_iota(jnp.int32, (1, D), 1)
    c_ref[...] = jnp.where(lane == 0, c1, 0.0) + jnp.where(lane == 1, c2, 0.0)


def _tc_d_body(z_ref, g_ref, c_ref, o_ref):
    c1 = c_ref[0, 0]
    c2 = c_ref[0, 1]
    g = g_ref[...]
    z1 = z_ref[0:1, :] + c1 * g + c2
    z2 = z_ref[1:2, :] + c1 * g + c2
    p1 = jax.nn.sigmoid(z1)
    p2 = jax.nn.sigmoid(z2)
    lp = jnp.maximum(jnp.log(p1), -100.0)
    l1p = jnp.maximum(jnp.log(1.0 - p2), -100.0)
    lane = lax.broadcasted_iota(jnp.int32, (1, NPAD), 1)
    valid = lane < N
    s1 = jnp.sum(jnp.where(valid, lp, 0.0))
    s2 = jnp.sum(jnp.where(valid, l1p, 0.0))
    o_ref[0, 0] = -0.5 * (s1 + s2) / N


def kernel(x, edge_index, W1, b1, W2, b2, Wd, perm):
    mesh = plsc.VectorSubcoreMesh(core_axis_name="c", subcore_axis_name="s",
                                  num_cores=1, num_subcores=NS)
    vec_t = jax.ShapeDtypeStruct((NPAD,), _f32)

    sc_a = pl.kernel(
        _sc_a_body,
        out_type=(vec_t, vec_t, vec_t),
        mesh=mesh,
        compiler_params=pltpu.CompilerParams(needs_layout_passes=False),
        scratch_types=[
            pltpu.VMEM((EPC,), _i32),
            pltpu.VMEM((EPC,), _i32),
            pltpu.VMEM((NPAD,), _f32),
            pltpu.VMEM((NPAD,), _f32),
            pltpu.VMEM((SLICE,), _f32),
            pltpu.VMEM((NS, SLICE), _f32),
            pltpu.VMEM((SLICE,), _f32),
            pltpu.VMEM((SLICE,), _f32),
            pltpu.VMEM((SLICE,), _f32),
            pltpu.VMEM_SHARED((NS, NPAD), _f32),
            pltpu.VMEM_SHARED((NPAD,), _f32),
            pltpu.SemaphoreType.DMA,
            pltpu.SemaphoreType.DMA,
        ],
    )
    ei_flat = edge_index.reshape(2 * E)
    dinv, r, q = sc_a(ei_flat)

    y2, cvec = pl.pallas_call(
        _tc_b_body,
        out_shape=[jax.ShapeDtypeStruct((NPAD, 1), _f32),
                   jax.ShapeDtypeStruct((1, D), _f32)],
    )(x, q[:N].reshape(N, 1), r[:N].reshape(N, 1), W1, W2, Wd,
      b1.reshape(1, D), b2.reshape(1, D))

    perm_pad = jnp.concatenate([perm.astype(_i32),
                                jnp.zeros((NPAD - N,), _i32)])

    sc_c = pl.kernel(
        _sc_c_body,
        out_type=(jax.ShapeDtypeStruct((2, NPAD), _f32), vec_t),
        mesh=mesh,
        compiler_params=pltpu.CompilerParams(needs_layout_passes=False),
        scratch_types=[
            pltpu.VMEM((EPC,), _i32),
            pltpu.VMEM((EPC,), _i32),
            pltpu.VMEM((3 * NPAD,), _f32),
            pltpu.VMEM((3 * NPAD,), _f32),
            pltpu.VMEM((SLICE,), _f32),
            pltpu.VMEM((NS, SLICE), _f32),
            pltpu.VMEM((SLICE,), _f32),
            pltpu.VMEM((SLICE,), _i32),
            pltpu.VMEM((SLICE,), _f32),
            pltpu.VMEM((SLICE,), _f32),
            pltpu.VMEM((SLICE,), _f32),
            pltpu.VMEM_SHARED((NS, NPAD), _f32),
            pltpu.VMEM_SHARED((2 * NPAD,), _f32),
            pltpu.SemaphoreType.DMA,
            pltpu.SemaphoreType.DMA,
        ],
    )
    z, g = sc_c(ei_flat, y2.reshape(NPAD), perm_pad, dinv)

    out = pl.pallas_call(
        _tc_d_body,
        out_shape=jax.ShapeDtypeStruct((1, 1), _f32),
        in_specs=[
            pl.BlockSpec(memory_space=pltpu.VMEM),
            pl.BlockSpec(memory_space=pltpu.VMEM),
            pl.BlockSpec(memory_space=pltpu.SMEM),
        ],
        out_specs=pl.BlockSpec(memory_space=pltpu.SMEM),
    )(z, g.reshape(1, NPAD), cvec)
    return out[0, 0]


# speedup vs baseline: 131.3196x; 1.0367x over previous
import jax
import jax.numpy as jnp
from jax import lax
from jax.experimental import pallas as pl
from jax.experimental.pallas import tpu as pltpu
from jax.experimental.pallas import tpu_sc as plsc

N = 10000
E = 320000
D = 128
NS = 16
L = 16
NPAD = 10240
SLICE = NPAD // NS
NV = SLICE // L
EPC = E // NS
EIT = EPC // L

_f32 = jnp.float32
_i32 = jnp.int32


def _zero_vec(ref, base, nvregs):
    zero16 = jnp.zeros((L,), _f32)

    @plsc.parallel_loop(0, nvregs, 1, unroll=8)
    def _zb(i):
        ref[pl.ds(base + i * L, L)] = zero16


def _reduce_partials(acc, nwords, S, red, stage16, col, sid, sbase):
    pltpu.sync_copy(S.at[:, pl.ds(col * NPAD + sbase, SLICE)], stage16)

    @plsc.parallel_loop(0, NV, 1, unroll=2)
    def _ab(j):
        t = stage16[0, pl.ds(j * L, L)]
        for k in range(1, NS):
            t = t + stage16[k, pl.ds(j * L, L)]
        red[pl.ds(j * L, L)] = t


def _rsqrt16(dv):
    magic = jnp.full((L,), 0x5F3759DF, _i32)
    ii = magic - lax.shift_right_logical(plsc.bitcast(dv, _i32), 1)
    yv = plsc.bitcast(ii, _f32)
    yv = yv * (1.5 - 0.5 * dv * yv * yv)
    yv = yv * (1.5 - 0.5 * dv * yv * yv)
    yv = yv * (1.5 - 0.5 * dv * yv * yv)
    return yv


def _sc_a_body(ei_hbm, dinv_hbm, r_hbm, q_hbm,
               src_v, dst_v, vec_a, acc, red, stage16, dinv_sl, a2_sl, out_sl,
               S, F, sem1, sem2):
    sid = lax.axis_index("s")
    ebase = sid * EPC
    sbase = sid * SLICE
    cp_s = pltpu.async_copy(ei_hbm.at[pl.ds(ebase, EPC)], src_v, sem1)
    cp_d = pltpu.async_copy(ei_hbm.at[pl.ds(E + ebase, EPC)], dst_v, sem2)

    one16 = jnp.ones((L,), _f32)

    _zero_vec(acc, 0, NPAD // L)
    cp_s.wait()
    cp_d.wait()

    @plsc.parallel_loop(0, EIT, 1, unroll=8)
    def _deg_b(i):
        di = dst_v[pl.ds(i * L, L)]
        plsc.addupdate_scatter(acc, [di], one16)

    pltpu.sync_copy(acc, S.at[sid])
    plsc.subcore_barrier()
    _reduce_partials(acc, NPAD, S, red, stage16, 0, sid, sbase)

    @plsc.parallel_loop(0, NV, 1)
    def _dv_b(j):
        dinv_sl[pl.ds(j * L, L)] = _rsqrt16(red[pl.ds(j * L, L)] + 1.0)

    plsc.subcore_barrier()
    pltpu.sync_copy(dinv_sl, F.at[pl.ds(sbase, SLICE)])
    pltpu.sync_copy(dinv_sl, dinv_hbm.at[pl.ds(sbase, SLICE)])
    plsc.subcore_barrier()
    pltpu.sync_copy(F, vec_a)

    def t_pass_loop():
        @plsc.parallel_loop(0, EIT, 1, unroll=8)
        def _t_b(i):
            si = src_v[pl.ds(i * L, L)]
            di = dst_v[pl.ds(i * L, L)]
            vals = plsc.load_gather(vec_a, [di])
            plsc.addupdate_scatter(acc, [si], vals)

    _zero_vec(acc, 0, NPAD // L)
    t_pass_loop()
    pltpu.sync_copy(acc, S.at[sid])
    plsc.subcore_barrier()
    _reduce_partials(acc, NPAD, S, red, stage16, 0, sid, sbase)

    @plsc.parallel_loop(0, NV, 1)
    def _rf_b(j):
        dsv = dinv_sl[pl.ds(j * L, L)]
        rr = dsv * (red[pl.ds(j * L, L)] + dsv)
        out_sl[pl.ds(j * L, L)] = rr
        a2_sl[pl.ds(j * L, L)] = dsv * rr

    pltpu.sync_copy(out_sl, r_hbm.at[pl.ds(sbase, SLICE)])
    plsc.subcore_barrier()
    pltpu.sync_copy(a2_sl, F.at[pl.ds(sbase, SLICE)])
    plsc.subcore_barrier()
    pltpu.sync_copy(F, vec_a)

    _zero_vec(acc, 0, NPAD // L)
    t_pass_loop()
    pltpu.sync_copy(acc, S.at[sid])
    plsc.subcore_barrier()
    _reduce_partials(acc, NPAD, S, red, stage16, 0, sid, sbase)

    @plsc.parallel_loop(0, NV, 1)
    def _qf_b(j):
        dsv = dinv_sl[pl.ds(j * L, L)]
        out_sl[pl.ds(j * L, L)] = dsv * (red[pl.ds(j * L, L)] + a2_sl[pl.ds(j * L, L)])

    pltpu.sync_copy(out_sl, q_hbm.at[pl.ds(sbase, SLICE)])


def _sc_c_body(ei_hbm, y_hbm, perm_hbm, dinv_hbm, z_hbm, g_hbm,
               src_v, dst_v, vecs, acc, red, stage16, dinv_sl, perm_sl,
               b1_sl, b2_sl, out_sl, S, F2, sem1, sem2):
    sid = lax.axis_index("s")
    ebase = sid * EPC
    sbase = sid * SLICE
    cp_s = pltpu.async_copy(ei_hbm.at[pl.ds(ebase, EPC)], src_v, sem1)
    cp_d = pltpu.async_copy(ei_hbm.at[pl.ds(E + ebase, EPC)], dst_v, sem2)
    pltpu.sync_copy(dinv_hbm.at[pl.ds(sbase, SLICE)], dinv_sl)

    TAIL = N - (NS - 1) * SLICE
    zero16i = jnp.zeros((L,), _i32)

    @pl.when(sid < NS - 1)
    def _():
        pltpu.sync_copy(perm_hbm.at[pl.ds(sbase, SLICE)], perm_sl)

    @pl.when(sid == NS - 1)
    def _():
        pltpu.sync_copy(perm_hbm.at[pl.ds((NS - 1) * SLICE, TAIL)],
                        perm_sl.at[pl.ds(0, TAIL)])
        for j in range(TAIL // L, NV):
            perm_sl[pl.ds(j * L, L)] = zero16i
    pltpu.sync_copy(y_hbm, vecs.at[pl.ds(0, NPAD)])
    pltpu.sync_copy(dinv_hbm, vecs.at[pl.ds(2 * NPAD, NPAD)])

    @plsc.parallel_loop(0, NV, 1, unroll=4)
    def _ac_b(j):
        pv = perm_sl[pl.ds(j * L, L)]
        yv = plsc.load_gather(vecs, [pv])
        out_sl[pl.ds(j * L, L)] = dinv_sl[pl.ds(j * L, L)] * yv

    pltpu.sync_copy(out_sl, F2.at[pl.ds(sbase, SLICE)])

    @plsc.parallel_loop(0, NPAD // L, 1, unroll=4)
    def _ar_b(j):
        vecs[pl.ds(j * L, L)] = (vecs[pl.ds(j * L, L)]
                                 * vecs[pl.ds(2 * NPAD + j * L, L)])

    plsc.subcore_barrier()
    pltpu.sync_copy(F2.at[pl.ds(0, NPAD)], vecs.at[pl.ds(NPAD, NPAD)])

    _zero_vec(acc, 0, 3 * NPAD // L)
    cp_s.wait()
    cp_d.wait()
    off1 = jnp.full((L,), NPAD, _i32)
    off2 = jnp.full((L,), 2 * NPAD, _i32)

    @plsc.parallel_loop(0, EIT, 1, unroll=4)
    def _h1_b(i):
        si = src_v[pl.ds(i * L, L)]
        di = dst_v[pl.ds(i * L, L)]
        v0 = plsc.load_gather(vecs, [si])
        v1 = plsc.load_gather(vecs, [si + off1])
        v2 = plsc.load_gather(vecs, [si + off2])
        plsc.addupdate_scatter(acc, [di], v0)
        plsc.addupdate_scatter(acc, [di + off1], v1)
        plsc.addupdate_scatter(acc, [di + off2], v2)

    def col_reduce(col):
        pltpu.sync_copy(acc.at[pl.ds(col * NPAD, NPAD)], S.at[sid])
        plsc.subcore_barrier()
        _reduce_partials(acc, NPAD, S, red, stage16, 0, sid, sbase)
        plsc.subcore_barrier()

    col_reduce(0)

    @plsc.parallel_loop(0, NV, 1)
    def _b1_b(j):
        dsv = dinv_sl[pl.ds(j * L, L)]
        av = vecs[pl.ds(sbase + j * L, L)]
        b1_sl[pl.ds(j * L, L)] = dsv * dsv * (red[pl.ds(j * L, L)] + av)

    col_reduce(1)

    @plsc.parallel_loop(0, NV, 1)
    def _b2_b(j):
        dsv = dinv_sl[pl.ds(j * L, L)]
        av = vecs[pl.ds(NPAD + sbase + j * L, L)]
        b2_sl[pl.ds(j * L, L)] = dsv * dsv * (red[pl.ds(j * L, L)] + av)

    col_reduce(2)

    @plsc.parallel_loop(0, NV, 1)
    def _g_b(j):
        dsv = dinv_sl[pl.ds(j * L, L)]
        out_sl[pl.ds(j * L, L)] = dsv * (red[pl.ds(j * L, L)] + dsv)

    pltpu.sync_copy(out_sl, g_hbm.at[pl.ds(sbase, SLICE)])
    pltpu.sync_copy(b1_sl, F2.at[pl.ds(sbase, SLICE)])
    pltpu.sync_copy(b2_sl, F2.at[pl.ds(NPAD + sbase, SLICE)])
    plsc.subcore_barrier()
    pltpu.sync_copy(F2, vecs.at[pl.ds(0, 2 * NPAD)])

    _zero_vec(acc, 0, 2 * NPAD // L)

    @plsc.parallel_loop(0, EIT, 1, unroll=4)
    def _h2_b(i):
        si = src_v[pl.ds(i * L, L)]
        di = dst_v[pl.ds(i * L, L)]
        v0 = plsc.load_gather(vecs, [si])
        v1 = plsc.load_gather(vecs, [si + off1])
        plsc.addupdate_scatter(acc, [di], v0)
        plsc.addupdate_scatter(acc, [di + off1], v1)

    col_reduce(0)

    @plsc.parallel_loop(0, NV, 1)
    def _z1_b(j):
        dsv = dinv_sl[pl.ds(j * L, L)]
        bv = b1_sl[pl.ds(j * L, L)]
        out_sl[pl.ds(j * L, L)] = dsv * (red[pl.ds(j * L, L)] + bv)

    pltpu.sync_copy(out_sl, z_hbm.at[0, pl.ds(sbase, SLICE)])

    col_reduce(1)

    @plsc.parallel_loop(0, NV, 1)
    def _z2_b(j):
        dsv = dinv_sl[pl.ds(j * L, L)]
        bv = b2_sl[pl.ds(j * L, L)]
        out_sl[pl.ds(j * L, L)] = dsv * (red[pl.ds(j * L, L)] + bv)

    pltpu.sync_copy(out_sl, z_hbm.at[1, pl.ds(sbase, SLICE)])


def _tc_b_body(x_ref, q_ref, r_ref, w1_ref, w2_ref, wd_ref, b1_ref, b2_ref,
               y_ref, c_ref):
    f32 = jnp.float32
    X = x_ref[...]
    q = q_ref[0:N, :]
    sum_r = jnp.sum(r_ref[0:N, :])
    qx = lax.dot_general(q, X, (((0,), (0,)), ((), ())),
                         preferred_element_type=f32)
    t1 = lax.dot_general(qx, w1_ref[...], (((1,), (1,)), ((), ())),
                         preferred_element_type=f32)
    m = lax.dot_general(t1 * (1.0 / N) + (sum_r / N) * b1_ref[...],
                        w2_ref[...], (((1,), (1,)), ((), ())),
                        preferred_element_type=f32) + b2_ref[...]
    s = jax.nn.sigmoid(m)
    v = lax.dot_general(s, wd_ref[...], (((1,), (0,)), ((), ())),
                        preferred_element_type=f32)
    u = lax.dot_general(v, w2_ref[...], (((1,), (0,)), ((), ())),
                        preferred_element_type=f32)
    w = lax.dot_general(u, w1_ref[...], (((1,), (0,)), ((), ())),
                        preferred_element_type=f32)
    y_ref[0:N, :] = lax.dot_general(X, w, (((1,), (1,)), ((), ())),
                                    preferred_element_type=f32)
    y_ref[N:NPAD, :] = jnp.zeros((NPAD - N, 1), f32)
    c1 = jnp.sum(b1_ref[...] * u)
    c2 = jnp.sum(b2_ref[...] * v)
    lane = lax.broadcasted_iota(jnp.int32, (1, D), 1)
    c_ref[...] = jnp.where(lane == 0, c1, 0.0) + jnp.where(lane == 1, c2, 0.0)


def _tc_d_body(z_ref, g_ref, c_ref, o_ref):
    c1 = c_ref[0, 0]
    c2 = c_ref[0, 1]
    g = g_ref[...]
    z1 = z_ref[0:1, :] + c1 * g + c2
    z2 = z_ref[1:2, :] + c1 * g + c2
    p1 = jax.nn.sigmoid(z1)
    p2 = jax.nn.sigmoid(z2)
    lp = jnp.maximum(jnp.log(p1), -100.0)
    l1p = jnp.maximum(jnp.log(1.0 - p2), -100.0)
    lane = lax.broadcasted_iota(jnp.int32, (1, NPAD), 1)
    valid = lane < N
    s1 = jnp.sum(jnp.where(valid, lp, 0.0))
    s2 = jnp.sum(jnp.where(valid, l1p, 0.0))
    o_ref[0, 0] = -0.5 * (s1 + s2) / N


def kernel(x, edge_index, W1, b1, W2, b2, Wd, perm):
    mesh = plsc.VectorSubcoreMesh(core_axis_name="c", subcore_axis_name="s",
                                  num_cores=1, num_subcores=NS)
    vec_t = jax.ShapeDtypeStruct((NPAD,), _f32)

    sc_a = pl.kernel(
        _sc_a_body,
        out_type=(vec_t, vec_t, vec_t),
        mesh=mesh,
        compiler_params=pltpu.CompilerParams(needs_layout_passes=False),
        scratch_types=[
            pltpu.VMEM((EPC,), _i32),
            pltpu.VMEM((EPC,), _i32),
            pltpu.VMEM((NPAD,), _f32),
            pltpu.VMEM((NPAD,), _f32),
            pltpu.VMEM((SLICE,), _f32),
            pltpu.VMEM((NS, SLICE), _f32),
            pltpu.VMEM((SLICE,), _f32),
            pltpu.VMEM((SLICE,), _f32),
            pltpu.VMEM((SLICE,), _f32),
            pltpu.VMEM_SHARED((NS, NPAD), _f32),
            pltpu.VMEM_SHARED((NPAD,), _f32),
            pltpu.SemaphoreType.DMA,
            pltpu.SemaphoreType.DMA,
        ],
    )
    ei_flat = edge_index.reshape(2 * E)
    dinv, r, q = sc_a(ei_flat)

    y2, cvec = pl.pallas_call(
        _tc_b_body,
        out_shape=[jax.ShapeDtypeStruct((NPAD, 1), _f32),
                   jax.ShapeDtypeStruct((1, D), _f32)],
    )(x, q.reshape(NPAD, 1), r.reshape(NPAD, 1), W1, W2, Wd,
      b1.reshape(1, D), b2.reshape(1, D))

    sc_c = pl.kernel(
        _sc_c_body,
        out_type=(jax.ShapeDtypeStruct((2, NPAD), _f32), vec_t),
        mesh=mesh,
        compiler_params=pltpu.CompilerParams(needs_layout_passes=False),
        scratch_types=[
            pltpu.VMEM((EPC,), _i32),
            pltpu.VMEM((EPC,), _i32),
            pltpu.VMEM((3 * NPAD,), _f32),
            pltpu.VMEM((3 * NPAD,), _f32),
            pltpu.VMEM((SLICE,), _f32),
            pltpu.VMEM((NS, SLICE), _f32),
            pltpu.VMEM((SLICE,), _f32),
            pltpu.VMEM((SLICE,), _i32),
            pltpu.VMEM((SLICE,), _f32),
            pltpu.VMEM((SLICE,), _f32),
            pltpu.VMEM((SLICE,), _f32),
            pltpu.VMEM_SHARED((NS, NPAD), _f32),
            pltpu.VMEM_SHARED((2 * NPAD,), _f32),
            pltpu.SemaphoreType.DMA,
            pltpu.SemaphoreType.DMA,
        ],
    )
    z, g = sc_c(ei_flat, y2.reshape(NPAD), perm.astype(_i32), dinv)

    out = pl.pallas_call(
        _tc_d_body,
        out_shape=jax.ShapeDtypeStruct((1, 1), _f32),
        in_specs=[
            pl.BlockSpec(memory_space=pltpu.VMEM),
            pl.BlockSpec(memory_space=pltpu.VMEM),
            pl.BlockSpec(memory_space=pltpu.SMEM),
        ],
        out_specs=pl.BlockSpec(memory_space=pltpu.SMEM),
    )(z, g.reshape(1, NPAD), cvec)
    return out[0, 0]


# speedup vs baseline: 132.5126x; 1.0091x over previous
import jax
import jax.numpy as jnp
from jax import lax
from jax.experimental import pallas as pl
from jax.experimental.pallas import tpu as pltpu
from jax.experimental.pallas import tpu_sc as plsc

N = 10000
E = 320000
D = 128
NS = 16
L = 16
NPAD = 10240
SLICE = NPAD // NS
NV = SLICE // L
EPC = E // NS
EIT = EPC // L

_f32 = jnp.float32
_i32 = jnp.int32


def _zero_vec(ref, base, nvregs):
    zero16 = jnp.zeros((L,), _f32)

    @plsc.parallel_loop(0, nvregs, 1, unroll=8)
    def _zb(i):
        ref[pl.ds(base + i * L, L)] = zero16


def _reduce_partials(acc, nwords, S, red, stage16, col, sid, sbase):
    pltpu.sync_copy(S.at[:, pl.ds(col * NPAD + sbase, SLICE)], stage16)

    @plsc.parallel_loop(0, NV, 1, unroll=2)
    def _ab(j):
        t = stage16[0, pl.ds(j * L, L)]
        for k in range(1, NS):
            t = t + stage16[k, pl.ds(j * L, L)]
        red[pl.ds(j * L, L)] = t


def _fastlog16(x):
    ii = plsc.bitcast(x, _i32)
    k = lax.shift_right_arithmetic(ii, jnp.full((L,), 23, _i32)) - 127
    m = plsc.bitcast(
        (ii & jnp.full((L,), 0x007FFFFF, _i32))
        | jnp.full((L,), 0x3F800000, _i32), _f32)
    t = (m - 1.0) / (m + 1.0)
    t2 = t * t
    ln_m = 2.0 * t * (1.0 + t2 * (1.0 / 3.0 + t2 * (0.2 + t2 * (1.0 / 7.0))))
    return k.astype(_f32) * 0.6931471805599453 + ln_m


def _rsqrt16(dv):
    magic = jnp.full((L,), 0x5F3759DF, _i32)
    ii = magic - lax.shift_right_logical(plsc.bitcast(dv, _i32), 1)
    yv = plsc.bitcast(ii, _f32)
    yv = yv * (1.5 - 0.5 * dv * yv * yv)
    yv = yv * (1.5 - 0.5 * dv * yv * yv)
    yv = yv * (1.5 - 0.5 * dv * yv * yv)
    return yv


def _sc_a_body(ei_hbm, dinv_hbm, r_hbm, q_hbm,
               src_v, dst_v, vec_a, acc, red, stage16, dinv_sl, a2_sl, out_sl,
               S, F, sem1, sem2):
    sid = lax.axis_index("s")
    ebase = sid * EPC
    sbase = sid * SLICE
    cp_s = pltpu.async_copy(ei_hbm.at[pl.ds(ebase, EPC)], src_v, sem1)
    cp_d = pltpu.async_copy(ei_hbm.at[pl.ds(E + ebase, EPC)], dst_v, sem2)

    one16 = jnp.ones((L,), _f32)

    _zero_vec(acc, 0, NPAD // L)
    cp_s.wait()
    cp_d.wait()

    @plsc.parallel_loop(0, EIT, 1, unroll=8)
    def _deg_b(i):
        di = dst_v[pl.ds(i * L, L)]
        plsc.addupdate_scatter(acc, [di], one16)

    pltpu.sync_copy(acc, S.at[sid])
    plsc.subcore_barrier()
    _reduce_partials(acc, NPAD, S, red, stage16, 0, sid, sbase)

    @plsc.parallel_loop(0, NV, 1)
    def _dv_b(j):
        dinv_sl[pl.ds(j * L, L)] = _rsqrt16(red[pl.ds(j * L, L)] + 1.0)

    plsc.subcore_barrier()
    pltpu.sync_copy(dinv_sl, F.at[pl.ds(sbase, SLICE)])
    pltpu.sync_copy(dinv_sl, dinv_hbm.at[pl.ds(sbase, SLICE)])
    plsc.subcore_barrier()
    pltpu.sync_copy(F, vec_a)

    def t_pass_loop():
        @plsc.parallel_loop(0, EIT, 1, unroll=8)
        def _t_b(i):
            si = src_v[pl.ds(i * L, L)]
            di = dst_v[pl.ds(i * L, L)]
            vals = plsc.load_gather(vec_a, [di])
            plsc.addupdate_scatter(acc, [si], vals)

    _zero_vec(acc, 0, NPAD // L)
    t_pass_loop()
    pltpu.sync_copy(acc, S.at[sid])
    plsc.subcore_barrier()
    _reduce_partials(acc, NPAD, S, red, stage16, 0, sid, sbase)

    @plsc.parallel_loop(0, NV, 1)
    def _rf_b(j):
        dsv = dinv_sl[pl.ds(j * L, L)]
        rr = dsv * (red[pl.ds(j * L, L)] + dsv)
        out_sl[pl.ds(j * L, L)] = rr
        a2_sl[pl.ds(j * L, L)] = dsv * rr

    pltpu.sync_copy(out_sl, r_hbm.at[pl.ds(sbase, SLICE)])
    plsc.subcore_barrier()
    pltpu.sync_copy(a2_sl, F.at[pl.ds(sbase, SLICE)])
    plsc.subcore_barrier()
    pltpu.sync_copy(F, vec_a)

    _zero_vec(acc, 0, NPAD // L)
    t_pass_loop()
    pltpu.sync_copy(acc, S.at[sid])
    plsc.subcore_barrier()
    _reduce_partials(acc, NPAD, S, red, stage16, 0, sid, sbase)

    @plsc.parallel_loop(0, NV, 1)
    def _qf_b(j):
        dsv = dinv_sl[pl.ds(j * L, L)]
        out_sl[pl.ds(j * L, L)] = dsv * (red[pl.ds(j * L, L)] + a2_sl[pl.ds(j * L, L)])

    pltpu.sync_copy(out_sl, q_hbm.at[pl.ds(sbase, SLICE)])


def _sc_c_body(ei_hbm, y_hbm, perm_hbm, dinv_hbm, c_hbm, loss_hbm,
               src_v, dst_v, vecs, acc, red, stage16, dinv_sl, perm_sl,
               b1_sl, b2_sl, out_sl, g_sl, cv16, buf16, S, F2, sem1, sem2):
    sid = lax.axis_index("s")
    ebase = sid * EPC
    sbase = sid * SLICE
    cp_s = pltpu.async_copy(ei_hbm.at[pl.ds(ebase, EPC)], src_v, sem1)
    cp_d = pltpu.async_copy(ei_hbm.at[pl.ds(E + ebase, EPC)], dst_v, sem2)
    pltpu.sync_copy(dinv_hbm.at[pl.ds(sbase, SLICE)], dinv_sl)
    pltpu.sync_copy(c_hbm.at[pl.ds(0, L)], cv16)
    lane16 = lax.broadcasted_iota(_i32, (L,), 0)
    cv = cv16[pl.ds(0, L)]
    c1 = jnp.sum(jnp.where(lane16 == 0, cv, 0.0))
    c2 = jnp.sum(jnp.where(lane16 == 1, cv, 0.0))

    TAIL = N - (NS - 1) * SLICE
    zero16i = jnp.zeros((L,), _i32)

    @pl.when(sid < NS - 1)
    def _():
        pltpu.sync_copy(perm_hbm.at[pl.ds(sbase, SLICE)], perm_sl)

    @pl.when(sid == NS - 1)
    def _():
        pltpu.sync_copy(perm_hbm.at[pl.ds((NS - 1) * SLICE, TAIL)],
                        perm_sl.at[pl.ds(0, TAIL)])
        for j in range(TAIL // L, NV):
            perm_sl[pl.ds(j * L, L)] = zero16i
    pltpu.sync_copy(y_hbm, vecs.at[pl.ds(0, NPAD)])
    pltpu.sync_copy(dinv_hbm, vecs.at[pl.ds(2 * NPAD, NPAD)])

    @plsc.parallel_loop(0, NV, 1, unroll=4)
    def _ac_b(j):
        pv = perm_sl[pl.ds(j * L, L)]
        yv = plsc.load_gather(vecs, [pv])
        out_sl[pl.ds(j * L, L)] = dinv_sl[pl.ds(j * L, L)] * yv

    pltpu.sync_copy(out_sl, F2.at[pl.ds(sbase, SLICE)])

    @plsc.parallel_loop(0, NPAD // L, 1, unroll=4)
    def _ar_b(j):
        vecs[pl.ds(j * L, L)] = (vecs[pl.ds(j * L, L)]
                                 * vecs[pl.ds(2 * NPAD + j * L, L)])

    plsc.subcore_barrier()
    pltpu.sync_copy(F2.at[pl.ds(0, NPAD)], vecs.at[pl.ds(NPAD, NPAD)])

    _zero_vec(acc, 0, 3 * NPAD // L)
    cp_s.wait()
    cp_d.wait()
    off1 = jnp.full((L,), NPAD, _i32)
    off2 = jnp.full((L,), 2 * NPAD, _i32)

    @plsc.parallel_loop(0, EIT, 1, unroll=4)
    def _h1_b(i):
        si = src_v[pl.ds(i * L, L)]
        di = dst_v[pl.ds(i * L, L)]
        v0 = plsc.load_gather(vecs, [si])
        v1 = plsc.load_gather(vecs, [si + off1])
        v2 = plsc.load_gather(vecs, [si + off2])
        plsc.addupdate_scatter(acc, [di], v0)
        plsc.addupdate_scatter(acc, [di + off1], v1)
        plsc.addupdate_scatter(acc, [di + off2], v2)

    def col_reduce(col):
        pltpu.sync_copy(acc.at[pl.ds(col * NPAD, NPAD)], S.at[sid])
        plsc.subcore_barrier()
        _reduce_partials(acc, NPAD, S, red, stage16, 0, sid, sbase)
        plsc.subcore_barrier()

    col_reduce(0)

    @plsc.parallel_loop(0, NV, 1)
    def _b1_b(j):
        dsv = dinv_sl[pl.ds(j * L, L)]
        av = vecs[pl.ds(sbase + j * L, L)]
        b1_sl[pl.ds(j * L, L)] = dsv * dsv * (red[pl.ds(j * L, L)] + av)

    col_reduce(1)

    @plsc.parallel_loop(0, NV, 1)
    def _b2_b(j):
        dsv = dinv_sl[pl.ds(j * L, L)]
        av = vecs[pl.ds(NPAD + sbase + j * L, L)]
        b2_sl[pl.ds(j * L, L)] = dsv * dsv * (red[pl.ds(j * L, L)] + av)

    col_reduce(2)

    @plsc.parallel_loop(0, NV, 1)
    def _g_b(j):
        dsv = dinv_sl[pl.ds(j * L, L)]
        g_sl[pl.ds(j * L, L)] = dsv * (red[pl.ds(j * L, L)] + dsv)

    pltpu.sync_copy(b1_sl, F2.at[pl.ds(sbase, SLICE)])
    pltpu.sync_copy(b2_sl, F2.at[pl.ds(NPAD + sbase, SLICE)])
    plsc.subcore_barrier()
    pltpu.sync_copy(F2, vecs.at[pl.ds(0, 2 * NPAD)])

    _zero_vec(acc, 0, 2 * NPAD // L)

    @plsc.parallel_loop(0, EIT, 1, unroll=4)
    def _h2_b(i):
        si = src_v[pl.ds(i * L, L)]
        di = dst_v[pl.ds(i * L, L)]
        v0 = plsc.load_gather(vecs, [si])
        v1 = plsc.load_gather(vecs, [si + off1])
        plsc.addupdate_scatter(acc, [di], v0)
        plsc.addupdate_scatter(acc, [di + off1], v1)

    nvalid = jnp.where(sid == NS - 1, TAIL // L, NV)

    def _zterm(j, b_ref):
        dsv = dinv_sl[pl.ds(j * L, L)]
        return (dsv * (red[pl.ds(j * L, L)] + b_ref[pl.ds(j * L, L)])
                + c1 * g_sl[pl.ds(j * L, L)] + c2)

    col_reduce(0)

    def _real_b(j, sv):
        p = 1.0 / (1.0 + jnp.exp(-_zterm(j, b1_sl)))
        return sv + jnp.maximum(_fastlog16(p), -100.0)

    sv = lax.fori_loop(0, nvalid, _real_b, jnp.zeros((L,), _f32))

    col_reduce(1)

    def _corr_b(j, sv2):
        p = 1.0 / (1.0 + jnp.exp(-_zterm(j, b2_sl)))
        return sv2 + jnp.maximum(_fastlog16(1.0 - p), -100.0)

    sv = lax.fori_loop(0, nvalid, _corr_b, sv)

    buf16[pl.ds(0, L)] = sv
    pltpu.sync_copy(buf16, F2.at[pl.ds(sid * L, L)])
    plsc.subcore_barrier()

    @pl.when(sid == 0)
    def _():
        pltpu.sync_copy(F2.at[pl.ds(0, NS * L)], red.at[pl.ds(0, NS * L)])
        tot = red[pl.ds(0, L)]
        for k in range(1, NS):
            tot = tot + red[pl.ds(k * L, L)]
        total = jnp.sum(tot)
        buf16[pl.ds(0, L)] = jnp.where(lane16 == 0, total * (-0.5 / N), 0.0)
        pltpu.sync_copy(buf16, loss_hbm)


def _tc_b_body(x_ref, q_ref, r_ref, w1_ref, w2_ref, wd_ref, b1_ref, b2_ref,
               y_ref, c_ref):
    f32 = jnp.float32
    X = x_ref[...]
    q = q_ref[0:N, :]
    sum_r = jnp.sum(r_ref[0:N, :])
    qx = lax.dot_general(q, X, (((0,), (0,)), ((), ())),
                         preferred_element_type=f32)
    t1 = lax.dot_general(qx, w1_ref[...], (((1,), (1,)), ((), ())),
                         preferred_element_type=f32)
    m = lax.dot_general(t1 * (1.0 / N) + (sum_r / N) * b1_ref[...],
                        w2_ref[...], (((1,), (1,)), ((), ())),
                        preferred_element_type=f32) + b2_ref[...]
    s = jax.nn.sigmoid(m)
    v = lax.dot_general(s, wd_ref[...], (((1,), (0,)), ((), ())),
                        preferred_element_type=f32)
    u = lax.dot_general(v, w2_ref[...], (((1,), (0,)), ((), ())),
                        preferred_element_type=f32)
    w = lax.dot_general(u, w1_ref[...], (((1,), (0,)), ((), ())),
                        preferred_element_type=f32)
    y_ref[0:N, :] = lax.dot_general(X, w, (((1,), (1,)), ((), ())),
                                    preferred_element_type=f32)
    y_ref[N:NPAD, :] = jnp.zeros((NPAD - N, 1), f32)
    c1 = jnp.sum(b1_ref[...] * u)
    c2 = jnp.sum(b2_ref[...] * v)
    lane = lax.broadcasted_iota(jnp.int32, (1, D), 1)
    c_ref[...] = jnp.where(lane == 0, c1, 0.0) + jnp.where(lane == 1, c2, 0.0)


def kernel(x, edge_index, W1, b1, W2, b2, Wd, perm):
    mesh = plsc.VectorSubcoreMesh(core_axis_name="c", subcore_axis_name="s",
                                  num_cores=1, num_subcores=NS)
    vec_t = jax.ShapeDtypeStruct((NPAD,), _f32)

    sc_a = pl.kernel(
        _sc_a_body,
        out_type=(vec_t, vec_t, vec_t),
        mesh=mesh,
        compiler_params=pltpu.CompilerParams(needs_layout_passes=False),
        scratch_types=[
            pltpu.VMEM((EPC,), _i32),
            pltpu.VMEM((EPC,), _i32),
            pltpu.VMEM((NPAD,), _f32),
            pltpu.VMEM((NPAD,), _f32),
            pltpu.VMEM((SLICE,), _f32),
            pltpu.VMEM((NS, SLICE), _f32),
            pltpu.VMEM((SLICE,), _f32),
            pltpu.VMEM((SLICE,), _f32),
            pltpu.VMEM((SLICE,), _f32),
            pltpu.VMEM_SHARED((NS, NPAD), _f32),
            pltpu.VMEM_SHARED((NPAD,), _f32),
            pltpu.SemaphoreType.DMA,
            pltpu.SemaphoreType.DMA,
        ],
    )
    ei_flat = edge_index.reshape(2 * E)
    dinv, r, q = sc_a(ei_flat)

    y2, cvec = pl.pallas_call(
        _tc_b_body,
        out_shape=[jax.ShapeDtypeStruct((NPAD, 1), _f32),
                   jax.ShapeDtypeStruct((1, D), _f32)],
    )(x, q.reshape(NPAD, 1), r.reshape(NPAD, 1), W1, W2, Wd,
      b1.reshape(1, D), b2.reshape(1, D))

    sc_c = pl.kernel(
        _sc_c_body,
        out_type=jax.ShapeDtypeStruct((L,), _f32),
        mesh=mesh,
        compiler_params=pltpu.CompilerParams(needs_layout_passes=False),
        scratch_types=[
            pltpu.VMEM((EPC,), _i32),
            pltpu.VMEM((EPC,), _i32),
            pltpu.VMEM((3 * NPAD,), _f32),
            pltpu.VMEM((3 * NPAD,), _f32),
            pltpu.VMEM((SLICE,), _f32),
            pltpu.VMEM((NS, SLICE), _f32),
            pltpu.VMEM((SLICE,), _f32),
            pltpu.VMEM((SLICE,), _i32),
            pltpu.VMEM((SLICE,), _f32),
            pltpu.VMEM((SLICE,), _f32),
            pltpu.VMEM((SLICE,), _f32),
            pltpu.VMEM((SLICE,), _f32),
            pltpu.VMEM((L,), _f32),
            pltpu.VMEM((L,), _f32),
            pltpu.VMEM_SHARED((NS, NPAD), _f32),
            pltpu.VMEM_SHARED((2 * NPAD,), _f32),
            pltpu.SemaphoreType.DMA,
            pltpu.SemaphoreType.DMA,
        ],
    )
    loss_vec = sc_c(ei_flat, y2.reshape(NPAD), perm.astype(_i32), dinv,
                    cvec.reshape(D))
    return loss_vec[0]


# speedup vs baseline: 132.9016x; 1.0029x over previous
import jax
import jax.numpy as jnp
from jax import lax
from jax.experimental import pallas as pl
from jax.experimental.pallas import tpu as pltpu
from jax.experimental.pallas import tpu_sc as plsc

N = 10000
E = 320000
D = 128
NS = 16
L = 16
NPAD = 10240
SLICE = NPAD // NS
NV = SLICE // L
EPC = E // NS
EIT = EPC // L

_f32 = jnp.float32
_i32 = jnp.int32


def _zero_vec(ref, base, nvregs):
    zero16 = jnp.zeros((L,), _f32)

    @plsc.parallel_loop(0, nvregs, 1, unroll=8)
    def _zb(i):
        ref[pl.ds(base + i * L, L)] = zero16


def _reduce_partials(acc, nwords, S, red, stage16, col, sid, sbase):
    pltpu.sync_copy(S.at[:, pl.ds(col * NPAD + sbase, SLICE)], stage16)

    @plsc.parallel_loop(0, NV, 1, unroll=2)
    def _ab(j):
        t = stage16[0, pl.ds(j * L, L)]
        for k in range(1, NS):
            t = t + stage16[k, pl.ds(j * L, L)]
        red[pl.ds(j * L, L)] = t


def _fastlog16(x):
    ii = plsc.bitcast(x, _i32)
    k = lax.shift_right_arithmetic(ii, jnp.full((L,), 23, _i32)) - 127
    m = plsc.bitcast(
        (ii & jnp.full((L,), 0x007FFFFF, _i32))
        | jnp.full((L,), 0x3F800000, _i32), _f32)
    t = (m - 1.0) / (m + 1.0)
    t2 = t * t
    ln_m = 2.0 * t * (1.0 + t2 * (1.0 / 3.0 + t2 * (0.2 + t2 * (1.0 / 7.0))))
    return k.astype(_f32) * 0.6931471805599453 + ln_m


def _rsqrt16(dv):
    magic = jnp.full((L,), 0x5F3759DF, _i32)
    ii = magic - lax.shift_right_logical(plsc.bitcast(dv, _i32), 1)
    yv = plsc.bitcast(ii, _f32)
    yv = yv * (1.5 - 0.5 * dv * yv * yv)
    yv = yv * (1.5 - 0.5 * dv * yv * yv)
    yv = yv * (1.5 - 0.5 * dv * yv * yv)
    return yv


def _sc_a_body(ei_hbm, dinv_hbm, r_hbm, q_hbm,
               src_v, dst_v, vec_a, acc, red, stage16, dinv_sl, a2_sl, out_sl,
               S, F, sem1, sem2):
    sid = lax.axis_index("s")
    ebase = sid * EPC
    sbase = sid * SLICE
    cp_s = pltpu.async_copy(ei_hbm.at[pl.ds(ebase, EPC)], src_v, sem1)
    cp_d = pltpu.async_copy(ei_hbm.at[pl.ds(E + ebase, EPC)], dst_v, sem2)

    one16 = jnp.ones((L,), _f32)

    _zero_vec(acc, 0, NPAD // L)
    cp_s.wait()
    cp_d.wait()

    @plsc.parallel_loop(0, EIT, 1, unroll=8)
    def _deg_b(i):
        di = dst_v[pl.ds(i * L, L)]
        plsc.addupdate_scatter(acc, [di], one16)

    pltpu.sync_copy(acc, S.at[sid])
    plsc.subcore_barrier()
    _reduce_partials(acc, NPAD, S, red, stage16, 0, sid, sbase)

    @plsc.parallel_loop(0, NV, 1)
    def _dv_b(j):
        dinv_sl[pl.ds(j * L, L)] = _rsqrt16(red[pl.ds(j * L, L)] + 1.0)

    plsc.subcore_barrier()
    pltpu.sync_copy(dinv_sl, F.at[pl.ds(sbase, SLICE)])
    pltpu.sync_copy(dinv_sl, dinv_hbm.at[pl.ds(sbase, SLICE)])
    plsc.subcore_barrier()
    pltpu.sync_copy(F, vec_a)

    def t_pass_loop():
        @plsc.parallel_loop(0, EIT, 1, unroll=8)
        def _t_b(i):
            si = src_v[pl.ds(i * L, L)]
            di = dst_v[pl.ds(i * L, L)]
            vals = plsc.load_gather(vec_a, [di])
            plsc.addupdate_scatter(acc, [si], vals)

    _zero_vec(acc, 0, NPAD // L)
    t_pass_loop()
    pltpu.sync_copy(acc, S.at[sid])
    plsc.subcore_barrier()
    _reduce_partials(acc, NPAD, S, red, stage16, 0, sid, sbase)

    @plsc.parallel_loop(0, NV, 1)
    def _rf_b(j):
        dsv = dinv_sl[pl.ds(j * L, L)]
        rr = dsv * (red[pl.ds(j * L, L)] + dsv)
        out_sl[pl.ds(j * L, L)] = rr
        a2_sl[pl.ds(j * L, L)] = dsv * rr

    pltpu.sync_copy(out_sl, r_hbm.at[pl.ds(sbase, SLICE)])
    plsc.subcore_barrier()
    pltpu.sync_copy(a2_sl, F.at[pl.ds(sbase, SLICE)])
    plsc.subcore_barrier()
    pltpu.sync_copy(F, vec_a)

    _zero_vec(acc, 0, NPAD // L)
    t_pass_loop()
    pltpu.sync_copy(acc, S.at[sid])
    plsc.subcore_barrier()
    _reduce_partials(acc, NPAD, S, red, stage16, 0, sid, sbase)

    @plsc.parallel_loop(0, NV, 1)
    def _qf_b(j):
        dsv = dinv_sl[pl.ds(j * L, L)]
        out_sl[pl.ds(j * L, L)] = dsv * (red[pl.ds(j * L, L)] + a2_sl[pl.ds(j * L, L)])

    pltpu.sync_copy(out_sl, q_hbm.at[pl.ds(sbase, SLICE)])


def _sc_c_body(ei_hbm, y_hbm, perm_hbm, dinv_hbm, c_hbm, loss_hbm,
               src_v, dst_v, vecs, acc, red, stage16, dinv_sl, perm_sl,
               b1_sl, b2_sl, out_sl, g_sl, cv16, buf16, S, F2, sem1, sem2):
    sid = lax.axis_index("s")
    ebase = sid * EPC
    sbase = sid * SLICE
    cp_s = pltpu.async_copy(ei_hbm.at[pl.ds(ebase, EPC)], src_v, sem1)
    cp_d = pltpu.async_copy(ei_hbm.at[pl.ds(E + ebase, EPC)], dst_v, sem2)
    pltpu.sync_copy(dinv_hbm.at[pl.ds(sbase, SLICE)], dinv_sl)
    pltpu.sync_copy(c_hbm.at[pl.ds(0, L)], cv16)
    lane16 = lax.broadcasted_iota(_i32, (L,), 0)
    cv = cv16[pl.ds(0, L)]
    c1 = jnp.sum(jnp.where(lane16 == 0, cv, 0.0))
    c2 = jnp.sum(jnp.where(lane16 == 1, cv, 0.0))

    TAIL = N - (NS - 1) * SLICE
    zero16i = jnp.zeros((L,), _i32)

    @pl.when(sid < NS - 1)
    def _():
        pltpu.sync_copy(perm_hbm.at[pl.ds(sbase, SLICE)], perm_sl)

    @pl.when(sid == NS - 1)
    def _():
        pltpu.sync_copy(perm_hbm.at[pl.ds((NS - 1) * SLICE, TAIL)],
                        perm_sl.at[pl.ds(0, TAIL)])
        for j in range(TAIL // L, NV):
            perm_sl[pl.ds(j * L, L)] = zero16i
    pltpu.sync_copy(y_hbm, vecs.at[pl.ds(0, NPAD)])
    pltpu.sync_copy(dinv_hbm, vecs.at[pl.ds(2 * NPAD, NPAD)])

    @plsc.parallel_loop(0, NV, 1, unroll=4)
    def _ac_b(j):
        pv = perm_sl[pl.ds(j * L, L)]
        yv = plsc.load_gather(vecs, [pv])
        out_sl[pl.ds(j * L, L)] = dinv_sl[pl.ds(j * L, L)] * yv

    pltpu.sync_copy(out_sl, F2.at[pl.ds(sbase, SLICE)])

    @plsc.parallel_loop(0, NPAD // L, 1, unroll=4)
    def _ar_b(j):
        vecs[pl.ds(j * L, L)] = (vecs[pl.ds(j * L, L)]
                                 * vecs[pl.ds(2 * NPAD + j * L, L)])

    plsc.subcore_barrier()
    pltpu.sync_copy(F2.at[pl.ds(0, NPAD)], vecs.at[pl.ds(NPAD, NPAD)])

    _zero_vec(acc, 0, 3 * NPAD // L)
    cp_s.wait()
    cp_d.wait()
    off1 = jnp.full((L,), NPAD, _i32)
    off2 = jnp.full((L,), 2 * NPAD, _i32)

    @plsc.parallel_loop(0, EIT, 1, unroll=8)
    def _h1_b(i):
        si = src_v[pl.ds(i * L, L)]
        di = dst_v[pl.ds(i * L, L)]
        v0 = plsc.load_gather(vecs, [si])
        v1 = plsc.load_gather(vecs, [si + off1])
        v2 = plsc.load_gather(vecs, [si + off2])
        plsc.addupdate_scatter(acc, [di], v0)
        plsc.addupdate_scatter(acc, [di + off1], v1)
        plsc.addupdate_scatter(acc, [di + off2], v2)

    def col_reduce(col):
        pltpu.sync_copy(acc.at[pl.ds(col * NPAD, NPAD)], S.at[sid])
        plsc.subcore_barrier()
        _reduce_partials(acc, NPAD, S, red, stage16, 0, sid, sbase)
        plsc.subcore_barrier()

    col_reduce(0)

    @plsc.parallel_loop(0, NV, 1)
    def _b1_b(j):
        dsv = dinv_sl[pl.ds(j * L, L)]
        av = vecs[pl.ds(sbase + j * L, L)]
        b1_sl[pl.ds(j * L, L)] = dsv * dsv * (red[pl.ds(j * L, L)] + av)

    col_reduce(1)

    @plsc.parallel_loop(0, NV, 1)
    def _b2_b(j):
        dsv = dinv_sl[pl.ds(j * L, L)]
        av = vecs[pl.ds(NPAD + sbase + j * L, L)]
        b2_sl[pl.ds(j * L, L)] = dsv * dsv * (red[pl.ds(j * L, L)] + av)

    col_reduce(2)

    @plsc.parallel_loop(0, NV, 1)
    def _g_b(j):
        dsv = dinv_sl[pl.ds(j * L, L)]
        g_sl[pl.ds(j * L, L)] = dsv * (red[pl.ds(j * L, L)] + dsv)

    pltpu.sync_copy(b1_sl, F2.at[pl.ds(sbase, SLICE)])
    pltpu.sync_copy(b2_sl, F2.at[pl.ds(NPAD + sbase, SLICE)])
    plsc.subcore_barrier()
    pltpu.sync_copy(F2, vecs.at[pl.ds(0, 2 * NPAD)])

    _zero_vec(acc, 0, 2 * NPAD // L)

    @plsc.parallel_loop(0, EIT, 1, unroll=8)
    def _h2_b(i):
        si = src_v[pl.ds(i * L, L)]
        di = dst_v[pl.ds(i * L, L)]
        v0 = plsc.load_gather(vecs, [si])
        v1 = plsc.load_gather(vecs, [si + off1])
        plsc.addupdate_scatter(acc, [di], v0)
        plsc.addupdate_scatter(acc, [di + off1], v1)

    nvalid = jnp.where(sid == NS - 1, TAIL // L, NV)

    def _zterm(j, b_ref):
        dsv = dinv_sl[pl.ds(j * L, L)]
        return (dsv * (red[pl.ds(j * L, L)] + b_ref[pl.ds(j * L, L)])
                + c1 * g_sl[pl.ds(j * L, L)] + c2)

    col_reduce(0)

    def _real_b(j, sv):
        p = 1.0 / (1.0 + jnp.exp(-_zterm(j, b1_sl)))
        return sv + jnp.maximum(_fastlog16(p), -100.0)

    sv = lax.fori_loop(0, nvalid, _real_b, jnp.zeros((L,), _f32))

    col_reduce(1)

    def _corr_b(j, sv2):
        p = 1.0 / (1.0 + jnp.exp(-_zterm(j, b2_sl)))
        return sv2 + jnp.maximum(_fastlog16(1.0 - p), -100.0)

    sv = lax.fori_loop(0, nvalid, _corr_b, sv)

    buf16[pl.ds(0, L)] = sv
    pltpu.sync_copy(buf16, F2.at[pl.ds(sid * L, L)])
    plsc.subcore_barrier()

    @pl.when(sid == 0)
    def _():
        pltpu.sync_copy(F2.at[pl.ds(0, NS * L)], red.at[pl.ds(0, NS * L)])
        tot = red[pl.ds(0, L)]
        for k in range(1, NS):
            tot = tot + red[pl.ds(k * L, L)]
        total = jnp.sum(tot)
        buf16[pl.ds(0, L)] = jnp.where(lane16 == 0, total * (-0.5 / N), 0.0)
        pltpu.sync_copy(buf16, loss_hbm)


def _tc_b_body(x_ref, q_ref, r_ref, w1_ref, w2_ref, wd_ref, b1_ref, b2_ref,
               y_ref, c_ref):
    f32 = jnp.float32
    X = x_ref[...]
    q = q_ref[0:N, :]
    sum_r = jnp.sum(r_ref[0:N, :])
    qx = lax.dot_general(q, X, (((0,), (0,)), ((), ())),
                         preferred_element_type=f32)
    t1 = lax.dot_general(qx, w1_ref[...], (((1,), (1,)), ((), ())),
                         preferred_element_type=f32)
    m = lax.dot_general(t1 * (1.0 / N) + (sum_r / N) * b1_ref[...],
                        w2_ref[...], (((1,), (1,)), ((), ())),
                        preferred_element_type=f32) + b2_ref[...]
    s = jax.nn.sigmoid(m)
    v = lax.dot_general(s, wd_ref[...], (((1,), (0,)), ((), ())),
                        preferred_element_type=f32)
    u = lax.dot_general(v, w2_ref[...], (((1,), (0,)), ((), ())),
                        preferred_element_type=f32)
    w = lax.dot_general(u, w1_ref[...], (((1,), (0,)), ((), ())),
                        preferred_element_type=f32)
    y_ref[0:N, :] = lax.dot_general(X, w, (((1,), (1,)), ((), ())),
                                    preferred_element_type=f32)
    y_ref[N:NPAD, :] = jnp.zeros((NPAD - N, 1), f32)
    c1 = jnp.sum(b1_ref[...] * u)
    c2 = jnp.sum(b2_ref[...] * v)
    lane = lax.broadcasted_iota(jnp.int32, (1, D), 1)
    c_ref[...] = jnp.where(lane == 0, c1, 0.0) + jnp.where(lane == 1, c2, 0.0)


def kernel(x, edge_index, W1, b1, W2, b2, Wd, perm):
    mesh = plsc.VectorSubcoreMesh(core_axis_name="c", subcore_axis_name="s",
                                  num_cores=1, num_subcores=NS)
    vec_t = jax.ShapeDtypeStruct((NPAD,), _f32)

    sc_a = pl.kernel(
        _sc_a_body,
        out_type=(vec_t, vec_t, vec_t),
        mesh=mesh,
        compiler_params=pltpu.CompilerParams(needs_layout_passes=False),
        scratch_types=[
            pltpu.VMEM((EPC,), _i32),
            pltpu.VMEM((EPC,), _i32),
            pltpu.VMEM((NPAD,), _f32),
            pltpu.VMEM((NPAD,), _f32),
            pltpu.VMEM((SLICE,), _f32),
            pltpu.VMEM((NS, SLICE), _f32),
            pltpu.VMEM((SLICE,), _f32),
            pltpu.VMEM((SLICE,), _f32),
            pltpu.VMEM((SLICE,), _f32),
            pltpu.VMEM_SHARED((NS, NPAD), _f32),
            pltpu.VMEM_SHARED((NPAD,), _f32),
            pltpu.SemaphoreType.DMA,
            pltpu.SemaphoreType.DMA,
        ],
    )
    ei_flat = edge_index.reshape(2 * E)
    dinv, r, q = sc_a(ei_flat)

    y2, cvec = pl.pallas_call(
        _tc_b_body,
        out_shape=[jax.ShapeDtypeStruct((NPAD, 1), _f32),
                   jax.ShapeDtypeStruct((1, D), _f32)],
    )(x, q.reshape(NPAD, 1), r.reshape(NPAD, 1), W1, W2, Wd,
      b1.reshape(1, D), b2.reshape(1, D))

    sc_c = pl.kernel(
        _sc_c_body,
        out_type=jax.ShapeDtypeStruct((L,), _f32),
        mesh=mesh,
        compiler_params=pltpu.CompilerParams(needs_layout_passes=False),
        scratch_types=[
            pltpu.VMEM((EPC,), _i32),
            pltpu.VMEM((EPC,), _i32),
            pltpu.VMEM((3 * NPAD,), _f32),
            pltpu.VMEM((3 * NPAD,), _f32),
            pltpu.VMEM((SLICE,), _f32),
            pltpu.VMEM((NS, SLICE), _f32),
            pltpu.VMEM((SLICE,), _f32),
            pltpu.VMEM((SLICE,), _i32),
            pltpu.VMEM((SLICE,), _f32),
            pltpu.VMEM((SLICE,), _f32),
            pltpu.VMEM((SLICE,), _f32),
            pltpu.VMEM((SLICE,), _f32),
            pltpu.VMEM((L,), _f32),
            pltpu.VMEM((L,), _f32),
            pltpu.VMEM_SHARED((NS, NPAD), _f32),
            pltpu.VMEM_SHARED((2 * NPAD,), _f32),
            pltpu.SemaphoreType.DMA,
            pltpu.SemaphoreType.DMA,
        ],
    )
    loss_vec = sc_c(ei_flat, y2.reshape(NPAD), perm.astype(_i32), dinv,
                    cvec.reshape(D))
    return loss_vec[0]
